# baseline (speedup 1.0000x reference)
import functools

import jax
import jax.numpy as jnp
from jax import lax
from jax.experimental import pallas as pl
from jax.experimental.pallas import tpu as pltpu
from jax.experimental.pallas import tpu_sc as plsc

F32 = jnp.float32
_CH = 128


def _sc_mesh():
    return plsc.VectorSubcoreMesh(core_axis_name="c", subcore_axis_name="s")


def _rup(x, m):
    return (x + m - 1) // m * m


def _silu(x):
    return x * (1.0 / (1.0 + jnp.exp(-x)))


def _sigmoid(x):
    return 1.0 / (1.0 + jnp.exp(-x))


def _gelu(x):
    return 0.5 * x * (1.0 + lax.erf(x * 0.7071067811865476))


def _ln(x, g, b, eps=1e-5):
    m = jnp.mean(x, axis=-1, keepdims=True)
    v = jnp.mean((x - m) ** 2, axis=-1, keepdims=True)
    return (x - m) * lax.rsqrt(v + eps) * g + b


def _dot(a, b):
    return jnp.dot(a, b, preferred_element_type=F32)


def _prep_body(n_real, x_ref, p_ref, w_ref, b_ref, g_ref, beta_ref, h_ref, pn_ref):
    npad = p_ref.shape[0]
    valid = (lax.broadcasted_iota(jnp.int32, (npad, 1), 0) < n_real).astype(F32)
    pos = p_ref[...]
    mean = jnp.sum(pos, axis=0, keepdims=True) * (1.0 / n_real)
    cen = (pos - mean) * valid
    rms = jnp.sqrt(jnp.sum(cen * cen) * (1.0 / n_real))
    pn_ref[...] = cen * (1.0 / jnp.maximum(rms, 1e-6))
    y = _dot(x_ref[...], w_ref[...]) + b_ref[...]
    h_ref[...] = _gelu(_ln(y, g_ref[...], beta_ref[...]))


def _prep(xp, posp, w, b, g, beta, n_real):
    npad = xp.shape[0]
    return pl.pallas_call(
        functools.partial(_prep_body, n_real),
        out_shape=(jax.ShapeDtypeStruct((npad, 128), F32),
                   jax.ShapeDtypeStruct((npad, 128), F32)),
    )(xp, posp, w, b, g, beta)


def _center_body(n_real, p_ref, o_ref):
    npad = p_ref.shape[0]
    valid = (lax.broadcasted_iota(jnp.int32, (npad, 1), 0) < n_real).astype(F32)
    pos = p_ref[...]
    mean = jnp.sum(pos * valid, axis=0, keepdims=True) * (1.0 / n_real)
    cen = (pos - mean) * valid
    rms = jnp.sqrt(jnp.sum(cen * cen) * (1.0 / n_real))
    o_ref[...] = cen * (1.0 / jnp.maximum(rms, 1e-6))


def _center(pos, n_real):
    return pl.pallas_call(
        functools.partial(_center_body, n_real),
        out_shape=jax.ShapeDtypeStruct(pos.shape, F32),
    )(pos)


def _gather4_body(h_ref, p_ref, r_ref, c_ref, hr_ref, hc_ref, pr_ref, pc_ref):
    bsz = hr_ref.shape[0]

    def body(i, _):
        r = r_ref[0, 0, i]
        c = c_ref[0, 0, i]
        hr_ref[pl.ds(i, 1), :] = h_ref[pl.ds(r, 1), :]
        hc_ref[pl.ds(i, 1), :] = h_ref[pl.ds(c, 1), :]
        pr_ref[pl.ds(i, 1), :] = p_ref[pl.ds(r, 1), :]
        pc_ref[pl.ds(i, 1), :] = p_ref[pl.ds(c, 1), :]
        return 0

    lax.fori_loop(0, bsz, body, 0)


def _gather4(h, pos, rowb, colb, eb):
    npad = h.shape[0]
    nb = rowb.shape[0]
    epad = nb * eb
    resident = lambda shp: pl.BlockSpec(shp, lambda i: (0, 0))
    idx_spec = pl.BlockSpec((1, 1, eb), lambda i: (i, 0, 0), memory_space=pltpu.SMEM)
    return pl.pallas_call(
        _gather4_body,
        grid=(nb,),
        in_specs=[resident((npad, 128)), resident((npad, 16)), idx_spec, idx_spec],
        out_specs=(pl.BlockSpec((eb, 128), lambda i: (i, 0)),
                   pl.BlockSpec((eb, 128), lambda i: (i, 0)),
                   pl.BlockSpec((eb, 16), lambda i: (i, 0)),
                   pl.BlockSpec((eb, 16), lambda i: (i, 0))),
        out_shape=(jax.ShapeDtypeStruct((epad, 128), F32),
                   jax.ShapeDtypeStruct((epad, 128), F32),
                   jax.ShapeDtypeStruct((epad, 16), F32),
                   jax.ShapeDtypeStruct((epad, 16), F32)),
    )(h, pos, rowb, colb)


def _gather2_body(p_ref, r_ref, c_ref, pr_ref, pc_ref):
    bsz = pr_ref.shape[0]

    def body(i, _):
        r = r_ref[0, 0, i]
        c = c_ref[0, 0, i]
        pr_ref[pl.ds(i, 1), :] = p_ref[pl.ds(r, 1), :]
        pc_ref[pl.ds(i, 1), :] = p_ref[pl.ds(c, 1), :]
        return 0

    lax.fori_loop(0, bsz, body, 0)


def _gather2(pos, rowb, colb, eb):
    npad = pos.shape[0]
    nb = rowb.shape[0]
    epad = nb * eb
    idx_spec = pl.BlockSpec((1, 1, eb), lambda i: (i, 0, 0), memory_space=pltpu.SMEM)
    return pl.pallas_call(
        _gather2_body,
        grid=(nb,),
        in_specs=[pl.BlockSpec((npad, 16), lambda i: (0, 0)), idx_spec, idx_spec],
        out_specs=(pl.BlockSpec((eb, 16), lambda i: (i, 0)),
                   pl.BlockSpec((eb, 16), lambda i: (i, 0))),
        out_shape=(jax.ShapeDtypeStruct((epad, 16), F32),
                   jax.ShapeDtypeStruct((epad, 16), F32)),
    )(pos, rowb, colb)


def _scat_pos_body(init_ref, df_ref, db_ref, r_ref, c_ref, out_ref):
    bsz = df_ref.shape[0]

    @pl.when(pl.program_id(0) == 0)
    def _():
        out_ref[...] = init_ref[...]

    def body(i, _):
        r = r_ref[0, 0, i]
        c = c_ref[0, 0, i]
        out_ref[pl.ds(r, 1), :] += df_ref[pl.ds(i, 1), :]
        out_ref[pl.ds(c, 1), :] += db_ref[pl.ds(i, 1), :]
        return 0

    lax.fori_loop(0, bsz, body, 0)


def _scatter_pos(init, df, db, rowb, colb, eb):
    npad = init.shape[0]
    nb = rowb.shape[0]
    idx_spec = pl.BlockSpec((1, 1, eb), lambda i: (i, 0, 0), memory_space=pltpu.SMEM)
    return pl.pallas_call(
        _scat_pos_body,
        grid=(nb,),
        in_specs=[pl.BlockSpec((npad, 16), lambda i: (0, 0)),
                  pl.BlockSpec((eb, 16), lambda i: (i, 0)),
                  pl.BlockSpec((eb, 16), lambda i: (i, 0)),
                  idx_spec, idx_spec],
        out_specs=pl.BlockSpec((npad, 16), lambda i: (0, 0)),
        out_shape=jax.ShapeDtypeStruct((npad, 16), F32),
    )(init, df, db, rowb, colb)


def _scat_msg_body(m_ref, r_ref, out_ref):
    bsz = m_ref.shape[0]

    @pl.when(pl.program_id(0) == 0)
    def _():
        out_ref[...] = jnp.zeros_like(out_ref)

    def body(i, _):
        r = r_ref[0, 0, i]
        out_ref[pl.ds(r, 1), :] += m_ref[pl.ds(i, 1), :]
        return 0

    lax.fori_loop(0, bsz, body, 0)


def _scatter_msg(m, rowb, npad, eb):
    nb = rowb.shape[0]
    idx_spec = pl.BlockSpec((1, 1, eb), lambda i: (i, 0, 0), memory_space=pltpu.SMEM)
    return pl.pallas_call(
        _scat_msg_body,
        grid=(nb,),
        in_specs=[pl.BlockSpec((eb, 128), lambda i: (i, 0)), idx_spec],
        out_specs=pl.BlockSpec((npad, 128), lambda i: (0, 0)),
        out_shape=jax.ShapeDtypeStruct((npad, 128), F32),
    )(m, rowb)


def _gc_gs_body(h_ref, r_ref, c_ref, out_ref):
    bsz = r_ref.shape[2]

    @pl.when(pl.program_id(0) == 0)
    def _():
        out_ref[...] = jnp.zeros_like(out_ref)

    def body(i, _):
        r = r_ref[0, 0, i]
        c = c_ref[0, 0, i]
        out_ref[pl.ds(c, 1), :] += h_ref[pl.ds(r, 1), :]
        return 0

    lax.fori_loop(0, bsz, body, 0)


def _gc_gather_scatter(h, rowb, colb, eb):
    npad = h.shape[0]
    nb = rowb.shape[0]
    idx_spec = pl.BlockSpec((1, 1, eb), lambda i: (i, 0, 0), memory_space=pltpu.SMEM)
    return pl.pallas_call(
        _gc_gs_body,
        grid=(nb,),
        in_specs=[pl.BlockSpec((npad, 128), lambda i: (0, 0)), idx_spec, idx_spec],
        out_specs=pl.BlockSpec((npad, 128), lambda i: (0, 0)),
        out_shape=jax.ShapeDtypeStruct((npad, 128), F32),
    )(h, rowb, colb)


def _sc_gather4(h, pos, row, col):
    npad = h.shape[0]
    epad = row.shape[0]
    ch = 64
    per_w = epad // 32
    nchunk = per_w // ch

    @functools.partial(
        pl.kernel,
        out_type=(jax.ShapeDtypeStruct((epad, 128), F32),
                  jax.ShapeDtypeStruct((epad, 128), F32),
                  jax.ShapeDtypeStruct((epad, 128), F32),
                  jax.ShapeDtypeStruct((epad, 128), F32)),
        mesh=_sc_mesh(),
        scratch_types=[pltpu.VMEM((per_w,), jnp.int32), pltpu.VMEM((per_w,), jnp.int32),
                       pltpu.VMEM((2, ch, 128), F32), pltpu.VMEM((2, ch, 128), F32),
                       pltpu.VMEM((2, ch, 128), F32), pltpu.VMEM((2, ch, 128), F32),
                       pltpu.SemaphoreType.DMA, pltpu.SemaphoreType.DMA],
    )
    def k(h_hbm, p_hbm, r_hbm, c_hbm, hr_hbm, hc_hbm, pr_hbm, pc_hbm,
          riv, civ, hrv, hcv, prv, pcv, sg0, sg1):
        wid = lax.axis_index("s") * 2 + lax.axis_index("c")
        base = wid * per_w
        sems = (sg0, sg1)

        pltpu.sync_copy(r_hbm.at[pl.ds(base, per_w)], riv)
        pltpu.sync_copy(c_hbm.at[pl.ds(base, per_w)], civ)

        def fire(j, b):
            ri = riv.at[pl.ds(j * ch, ch)]
            ci = civ.at[pl.ds(j * ch, ch)]
            pltpu.async_copy(h_hbm.at[ri], hrv.at[b], sems[b])
            pltpu.async_copy(h_hbm.at[ci], hcv.at[b], sems[b])
            pltpu.async_copy(p_hbm.at[ri], prv.at[b], sems[b])
            pltpu.async_copy(p_hbm.at[ci], pcv.at[b], sems[b])

        def drain_write(j, b):
            ri = riv.at[pl.ds(j * ch, ch)]
            ci = civ.at[pl.ds(j * ch, ch)]
            pltpu.make_async_copy(h_hbm.at[ri], hrv.at[b], sems[b]).wait()
            pltpu.make_async_copy(h_hbm.at[ci], hcv.at[b], sems[b]).wait()
            pltpu.make_async_copy(p_hbm.at[ri], prv.at[b], sems[b]).wait()
            pltpu.make_async_copy(p_hbm.at[ci], pcv.at[b], sems[b]).wait()
            off = base + j * ch
            pltpu.sync_copy(hrv.at[b], hr_hbm.at[pl.ds(off, ch)])
            pltpu.sync_copy(hcv.at[b], hc_hbm.at[pl.ds(off, ch)])
            pltpu.sync_copy(prv.at[b], pr_hbm.at[pl.ds(off, ch)])
            pltpu.sync_copy(pcv.at[b], pc_hbm.at[pl.ds(off, ch)])

        fire(0, 0)

        def body(st, _):
            j0 = 2 * st
            fire(j0 + 1, 1)
            drain_write(j0, 0)

            @pl.when(j0 + 2 < nchunk)
            def _():
                fire(j0 + 2, 0)

            drain_write(j0 + 1, 1)
            return 0

        lax.fori_loop(0, nchunk // 2, body, 0)

    return k(h, pos, row, col)


def _sc_gather_pos2(pos, row, col):
    epad = row.shape[0]
    ch = _CH
    per_w = epad // 32
    nchunk = per_w // ch

    @functools.partial(
        pl.kernel,
        out_type=(jax.ShapeDtypeStruct((epad, 128), F32),
                  jax.ShapeDtypeStruct((epad, 128), F32)),
        mesh=_sc_mesh(),
        scratch_types=[pltpu.VMEM((per_w,), jnp.int32), pltpu.VMEM((per_w,), jnp.int32),
                       pltpu.VMEM((2, ch, 128), F32), pltpu.VMEM((2, ch, 128), F32),
                       pltpu.SemaphoreType.DMA, pltpu.SemaphoreType.DMA],
    )
    def k(p_hbm, r_hbm, c_hbm, pr_hbm, pc_hbm, riv, civ, prv, pcv, sg0, sg1):
        wid = lax.axis_index("s") * 2 + lax.axis_index("c")
        base = wid * per_w
        sems = (sg0, sg1)

        pltpu.sync_copy(r_hbm.at[pl.ds(base, per_w)], riv)
        pltpu.sync_copy(c_hbm.at[pl.ds(base, per_w)], civ)

        def fire(j, b):
            ri = riv.at[pl.ds(j * ch, ch)]
            ci = civ.at[pl.ds(j * ch, ch)]
            pltpu.async_copy(p_hbm.at[ri], prv.at[b], sems[b])
            pltpu.async_copy(p_hbm.at[ci], pcv.at[b], sems[b])

        def drain_write(j, b):
            ri = riv.at[pl.ds(j * ch, ch)]
            ci = civ.at[pl.ds(j * ch, ch)]
            pltpu.make_async_copy(p_hbm.at[ri], prv.at[b], sems[b]).wait()
            pltpu.make_async_copy(p_hbm.at[ci], pcv.at[b], sems[b]).wait()
            off = base + j * ch
            pltpu.sync_copy(prv.at[b], pr_hbm.at[pl.ds(off, ch)])
            pltpu.sync_copy(pcv.at[b], pc_hbm.at[pl.ds(off, ch)])

        fire(0, 0)

        def body(st, _):
            j0 = 2 * st
            fire(j0 + 1, 1)
            drain_write(j0, 0)

            @pl.when(j0 + 2 < nchunk)
            def _():
                fire(j0 + 2, 0)

            drain_write(j0 + 1, 1)
            return 0

        lax.fori_loop(0, nchunk // 2, body, 0)

    return k(pos, row, col)


def _sc_scatter_pos(init, df, db, row, col):
    npad = init.shape[0]
    epad = row.shape[0]
    ch = _CH
    per_w = epad // 16
    nchunk = per_w // ch
    rps = npad // 16

    @functools.partial(
        pl.kernel,
        out_type=jax.ShapeDtypeStruct((npad, 128), F32),
        mesh=_sc_mesh(),
        scratch_types=[pltpu.VMEM((ch,), jnp.int32),
                       pltpu.VMEM((ch, 128), F32), pltpu.VMEM((ch, 128), F32),
                       pltpu.VMEM_SHARED((npad, 128), F32),
                       pltpu.SemaphoreType.DMA, pltpu.SemaphoreType.DMA],
    )
    def k(init_hbm, df_hbm, db_hbm, r_hbm, c_hbm, out_hbm,
          iv, vv0, vv1, acc, sg0, sg1):
        cid = lax.axis_index("c")
        sid = lax.axis_index("s")
        sems = (sg0, sg1)
        vvs = (vv0, vv1)

        @pl.when(cid == 0)
        def _():
            pltpu.sync_copy(init_hbm.at[pl.ds(sid * rps, rps)],
                            acc.at[pl.ds(sid * rps, rps)])
            plsc.subcore_barrier()
            base = sid * per_w

            def one_pass(v_hbm, i_hbm):
                def fire(j, b):
                    pltpu.async_copy(v_hbm.at[pl.ds(base + j * ch, ch)], vvs[b],
                                     sems[b])

                def drain_add(j, b):
                    pltpu.sync_copy(i_hbm.at[pl.ds(base + j * ch, ch)], iv)
                    pltpu.make_async_copy(v_hbm.at[pl.ds(base + j * ch, ch)],
                                          vvs[b], sems[b]).wait()
                    pltpu.sync_copy(vvs[b], acc.at[iv], add=True)

                fire(0, 0)

                def body(st, _):
                    j0 = 2 * st
                    fire(j0 + 1, 1)
                    drain_add(j0, 0)

                    @pl.when(j0 + 2 < nchunk)
                    def _():
                        fire(j0 + 2, 0)

                    drain_add(j0 + 1, 1)
                    return 0

                lax.fori_loop(0, nchunk // 2, body, 0)

            one_pass(df_hbm, r_hbm)
            one_pass(db_hbm, c_hbm)
            plsc.subcore_barrier()
            pltpu.sync_copy(acc.at[pl.ds(sid * rps, rps)],
                            out_hbm.at[pl.ds(sid * rps, rps)])

    return k(init, df, db, row, col)


def _sc_scatter_msg(m, row2d, zeros):
    npad = zeros.shape[0]
    ch = _CH
    nrow = row2d.shape[0]
    nchunk = nrow // 32
    per_w = nchunk * ch
    rps = npad // 16

    @functools.partial(
        pl.kernel,
        out_type=jax.ShapeDtypeStruct((2 * npad, 128), F32),
        mesh=_sc_mesh(),
        scratch_types=[pltpu.VMEM((nchunk, ch), jnp.int32),
                       pltpu.VMEM((ch, 128), F32), pltpu.VMEM((ch, 128), F32),
                       pltpu.VMEM_SHARED((npad, 128), F32),
                       pltpu.SemaphoreType.DMA, pltpu.SemaphoreType.DMA],
    )
    def k(m_hbm, r_hbm, z_hbm, out_hbm, riv, vv0, vv1, acc, sg0, sg1):
        cid = lax.axis_index("c")
        sid = lax.axis_index("s")
        sems = (sg0, sg1)
        vvs = (vv0, vv1)
        wid = sid * 2 + cid
        pltpu.sync_copy(z_hbm.at[pl.ds(sid * rps, rps)], acc.at[pl.ds(sid * rps, rps)])
        pltpu.sync_copy(r_hbm.at[pl.ds(wid * nchunk, nchunk)], riv)
        plsc.subcore_barrier()
        base = wid * per_w

        def fire(j, b):
            pltpu.async_copy(m_hbm.at[pl.ds(base + j * ch, ch)], vvs[b], sems[b])

        def drain_add(j, b):
            pltpu.make_async_copy(m_hbm.at[pl.ds(base + j * ch, ch)], vvs[b],
                                  sems[b]).wait()
            pltpu.sync_copy(vvs[b], acc.at[riv.at[j]], add=True)

        fire(0, 0)

        def body(st, _):
            j0 = 2 * st
            fire(j0 + 1, 1)
            drain_add(j0, 0)

            @pl.when(j0 + 2 < nchunk)
            def _():
                fire(j0 + 2, 0)

            drain_add(j0 + 1, 1)
            return 0

        lax.fori_loop(0, nchunk // 2, body, 0)
        plsc.subcore_barrier()
        pltpu.sync_copy(acc.at[pl.ds(sid * rps, rps)],
                        out_hbm.at[pl.ds(cid * npad + sid * rps, rps)])

    return k(m, row2d, zeros)


def _sc_gc(h, row, col2d, zeros):
    npad = h.shape[0]
    epad = row.shape[0]
    ch = _CH
    per_w = epad // 32
    nchunk = per_w // ch
    rps = npad // 16

    @functools.partial(
        pl.kernel,
        out_type=jax.ShapeDtypeStruct((2 * npad, 128), F32),
        mesh=_sc_mesh(),
        scratch_types=[pltpu.VMEM((per_w,), jnp.int32),
                       pltpu.VMEM((nchunk, ch), jnp.int32),
                       pltpu.VMEM((ch, 128), F32), pltpu.VMEM((ch, 128), F32),
                       pltpu.VMEM_SHARED((npad, 128), F32),
                       pltpu.SemaphoreType.DMA, pltpu.SemaphoreType.DMA],
    )
    def k(h_hbm, r_hbm, c_hbm, z_hbm, out_hbm, riv, civ, vv0, vv1, acc, sg0, sg1):
        cid = lax.axis_index("c")
        sid = lax.axis_index("s")
        sems = (sg0, sg1)
        vvs = (vv0, vv1)
        wid = sid * 2 + cid
        pltpu.sync_copy(z_hbm.at[pl.ds(sid * rps, rps)], acc.at[pl.ds(sid * rps, rps)])
        pltpu.sync_copy(r_hbm.at[pl.ds(wid * per_w, per_w)], riv)
        pltpu.sync_copy(c_hbm.at[pl.ds(wid * nchunk, nchunk)], civ)
        plsc.subcore_barrier()

        def fire(j, b):
            ri = riv.at[pl.ds(j * ch, ch)]
            pltpu.async_copy(h_hbm.at[ri], vvs[b], sems[b])

        def drain_add(j, b):
            ri = riv.at[pl.ds(j * ch, ch)]
            pltpu.make_async_copy(h_hbm.at[ri], vvs[b], sems[b]).wait()
            pltpu.sync_copy(vvs[b], acc.at[civ.at[j]], add=True)

        fire(0, 0)

        def body(st, _):
            j0 = 2 * st
            fire(j0 + 1, 1)
            drain_add(j0, 0)

            @pl.when(j0 + 2 < nchunk)
            def _():
                fire(j0 + 2, 0)

            drain_add(j0 + 1, 1)
            return 0

        lax.fori_loop(0, nchunk // 2, body, 0)
        plsc.subcore_barrier()
        pltpu.sync_copy(acc.at[pl.ds(sid * rps, rps)],
                        out_hbm.at[pl.ds(cid * npad + sid * rps, rps)])

    return k(h, row, col2d, zeros)


def _coord_body(hr_ref, hc_ref, pr_ref, pc_ref,
                w1a_ref, w1b_ref, w1c_ref, b1_ref, w2_ref, b2_ref,
                w3_ref, b3_ref, ew1_ref, ewb1_ref, ew2r_ref, ewb2_ref,
                scale_ref, df_ref, db_ref):
    hr = hr_ref[...]
    hc = hc_ref[...]
    rel = pr_ref[...] - pc_ref[...]
    w1a = w1a_ref[...]
    w1b = w1b_ref[...]
    w1c = w1c_ref[...]
    b1 = b1_ref[...]
    ha = _dot(hr, w1a)
    hb = _dot(hc, w1b)
    hab = _dot(hc, w1a)
    hbb = _dot(hr, w1b)
    rc = _dot(rel, w1c)
    t1 = _silu(ha + hb + rc + b1)
    u1 = _silu(hab + hbb - rc + b1)
    t2 = _silu(_dot(t1, w2_ref[...]) + b2_ref[...])
    u2 = _silu(_dot(u1, w2_ref[...]) + b2_ref[...])
    raw_f = jnp.tanh(_dot(t2, w3_ref[...]) + b3_ref[...])
    raw_b = jnp.tanh(_dot(u2, w3_ref[...]) + b3_ref[...])
    edge_len = jnp.sqrt(jnp.sum(rel * rel, axis=-1, keepdims=True))
    s = jnp.clip(scale_ref[0, 0], 0.0, 5.0)
    nf = jnp.maximum(jnp.sqrt(jnp.sum(raw_f * raw_f, axis=-1, keepdims=True)), 1e-8)
    nb_ = jnp.maximum(jnp.sqrt(jnp.sum(raw_b * raw_b, axis=-1, keepdims=True)), 1e-8)
    ew1 = ew1_ref[...]
    ewb1 = ewb1_ref[...]
    ew2r = ew2r_ref[...]
    ewb2 = ewb2_ref[0, 0]
    ew_f = _sigmoid(jnp.sum(_silu(_dot(rel, ew1) + ewb1) * ew2r, axis=-1, keepdims=True) + ewb2)
    ew_b = _sigmoid(jnp.sum(_silu(_dot(-rel, ew1) + ewb1) * ew2r, axis=-1, keepdims=True) + ewb2)
    common = 0.05 * s * edge_len
    df_ref[...] = raw_f / nf * (common * ew_f)
    db_ref[...] = raw_b / nb_ * (common * ew_b)


def _coord_pass(hr, hc, pr, pc, wp, eb):
    epad = hr.shape[0]
    nb = epad // eb
    ebspec = lambda d: pl.BlockSpec((eb, d), lambda i: (i, 0))
    wspec = lambda a: pl.BlockSpec(a.shape, lambda i: (0,) * a.ndim)
    weights = (wp['cw1a'], wp['cw1b'], wp['cw1c'], wp['cb1'], wp['cw2'], wp['cb2'],
               wp['cw3'], wp['cb3'], wp['ew1'], wp['ewb1'], wp['ew2r'], wp['ewb2'],
               wp['scale'])
    return pl.pallas_call(
        _coord_body,
        grid=(nb,),
        in_specs=[ebspec(128), ebspec(128), ebspec(128), ebspec(128)] +
                 [wspec(a) for a in weights],
        out_specs=(ebspec(128), ebspec(128)),
        out_shape=(jax.ShapeDtypeStruct((epad, 128), F32),
                   jax.ShapeDtypeStruct((epad, 128), F32)),
    )(hr, hc, pr, pc, *weights)


def _msg_body(hr_ref, hc_ref, pr_ref, pc_ref,
              w1a_ref, w1b_ref, w1c_ref, b1_ref, g1_ref, be1_ref,
              w2_ref, b2_ref, g2_ref, be2_ref, m_ref):
    rel = pr_ref[...] - pc_ref[...]
    m1 = _silu(_dot(hr_ref[...], w1a_ref[...]) + _dot(hc_ref[...], w1b_ref[...]) +
               _dot(rel, w1c_ref[...]) + b1_ref[...])
    m1 = _ln(m1, g1_ref[...], be1_ref[...])
    m2 = _silu(_dot(m1, w2_ref[...]) + b2_ref[...])
    m_ref[...] = _ln(m2, g2_ref[...], be2_ref[...])


def _msg_pass(hr, hc, pr2, pc2, wp, eb):
    epad = hr.shape[0]
    nb = epad // eb
    ebspec = lambda d: pl.BlockSpec((eb, d), lambda i: (i, 0))
    wspec = lambda a: pl.BlockSpec(a.shape, lambda i: (0,) * a.ndim)
    weights = (wp['mw1a'], wp['mw1b'], wp['mw1c'], wp['mb1'], wp['ln1g'], wp['ln1b'],
               wp['mw2'], wp['mb2'], wp['ln2g'], wp['ln2b'])
    return pl.pallas_call(
        _msg_body,
        grid=(nb,),
        in_specs=[ebspec(128), ebspec(128), ebspec(128), ebspec(128)] +
                 [wspec(a) for a in weights],
        out_specs=ebspec(128),
        out_shape=jax.ShapeDtypeStruct((epad, 128), F32),
    )(hr, hc, pr2, pc2, *weights)


def _node_body(h_ref, agg_ref, w1a_ref, w1b_ref, b1_ref, lg_ref, lb_ref,
               w2_ref, b2_ref, g_ref, be_ref, out_ref):
    h = h_ref[...]
    npad = h_ref.shape[0]
    agg = agg_ref[pl.ds(0, npad), :] + agg_ref[pl.ds(npad, npad), :]
    nm = _silu(_dot(h, w1a_ref[...]) + _dot(agg, w1b_ref[...]) + b1_ref[...])
    nm = _ln(nm, lg_ref[...], lb_ref[...])
    nm = _dot(nm, w2_ref[...]) + b2_ref[...]
    out_ref[...] = _ln(h + nm, g_ref[...], be_ref[...])


def _node_pass(h, agg, wp):
    npad = h.shape[0]
    weights = (wp['nw1a'], wp['nw1b'], wp['nb1'], wp['nlng'], wp['nlnb'],
               wp['nw2'], wp['nb2'], wp['lng'], wp['lnb'])
    return pl.pallas_call(
        _node_body,
        out_shape=jax.ShapeDtypeStruct((npad, 128), F32),
    )(h, agg, *weights)


def _gc_body(agg_ref, h_ref, rw_ref, rb_ref, rootw_ref, out_ref):
    npad = h_ref.shape[0]
    agg = agg_ref[pl.ds(0, npad), :] + agg_ref[pl.ds(npad, npad), :]
    out_ref[...] = (_dot(agg, rw_ref[...]) + rb_ref[...] +
                    _dot(h_ref[...], rootw_ref[...]))


def _gc_pass(agg, h, rw, rb, rootw):
    npad = h.shape[0]
    return pl.pallas_call(
        _gc_body,
        out_shape=jax.ShapeDtypeStruct((npad, 128), F32),
    )(agg, h, rw, rb, rootw)


def _final_body(n_real, n_groups, h_ref, batch_ref,
                gw1_ref, gb1_ref, glg_ref, glb_ref, gw2_ref, gb2_ref,
                gw3r_ref, gb3_ref, ow1_ref, ob1_ref, olg_ref, olb_ref,
                ow2_ref, ob2_ref, out_ref):
    npad = h_ref.shape[0]
    h = h_ref[...]
    gate = _dot(h, gw1_ref[...]) + gb1_ref[...]
    gate = _gelu(_ln(gate, glg_ref[...], glb_ref[...]))
    gate = _gelu(_dot(gate, gw2_ref[...]) + gb2_ref[...])
    gate_s = jnp.sum(gate * gw3r_ref[...], axis=-1, keepdims=True) + gb3_ref[0, 0]
    valid = lax.broadcasted_iota(jnp.int32, (npad, 1), 0) < n_real
    gid = lax.broadcasted_iota(jnp.int32, (1, n_groups), 1)
    oh = jnp.logical_and(batch_ref[...] == gid, valid).astype(F32)
    gmax = jnp.max(jnp.where(oh > 0, gate_s, -1e30), axis=0, keepdims=True)
    gmax_g = jnp.sum(oh * gmax, axis=-1, keepdims=True)
    ex = jnp.where(valid, jnp.exp(gate_s - gmax_g), 0.0)
    den = jnp.sum(oh * ex, axis=0, keepdims=True)
    den_g = jnp.sum(oh * den, axis=-1, keepdims=True)
    attn = ex / jnp.maximum(den_g, 1e-16)
    pooled = lax.dot_general(oh, attn * h, (((0,), (0,)), ((), ())),
                             preferred_element_type=F32)
    o = _gelu(_ln(_dot(pooled, ow1_ref[...]) + ob1_ref[...], olg_ref[...], olb_ref[...]))
    out_ref[...] = _dot(o, ow2_ref[...]) + ob2_ref[...]


def _final_pass(h, batchp, wp, n_real, n_groups, out_dim):
    weights = (wp['gw1'], wp['gb1'], wp['glg'], wp['glb'], wp['gw2'], wp['gb2'],
               wp['gw3r'], wp['gb3'], wp['ow1'], wp['ob1'], wp['olg'], wp['olb'],
               wp['ow2'], wp['ob2'])
    return pl.pallas_call(
        functools.partial(_final_body, n_real, n_groups),
        out_shape=jax.ShapeDtypeStruct((n_groups, out_dim), F32),
    )(h, batchp, *weights)


def _prep_egcl(p):
    pad_rows = lambda w, r: jnp.concatenate(
        [w, jnp.zeros((r - w.shape[0], w.shape[1]), F32)], axis=0)
    pad_cols = lambda w, c: jnp.concatenate(
        [w, jnp.zeros((w.shape[0], c - w.shape[1]), F32)], axis=1)
    row = lambda v: v.reshape(1, -1)
    return {
        'cw1a': p['coord_w1'][:128], 'cw1b': p['coord_w1'][128:256],
        'cw1c': pad_rows(p['coord_w1'][256:], 128), 'cb1': row(p['coord_b1']),
        'cw2': p['coord_w2'], 'cb2': row(p['coord_b2']),
        'cw3': pad_cols(p['coord_w3'], 128), 'cb3': pad_cols(row(p['coord_b3']), 128),
        'ew1': pad_rows(p['ew_w1'], 128), 'ewb1': row(p['ew_b1']),
        'ew2r': p['ew_w2'].reshape(1, -1), 'ewb2': p['ew_b2'].reshape(1, 1),
        'scale': p['scale'].reshape(1, 1),
        'mw1a': p['edge_w1'][:128], 'mw1b': p['edge_w1'][128:256],
        'mw1c': pad_rows(p['edge_w1'][256:], 128), 'mb1': row(p['edge_b1']),
        'ln1g': row(p['edge_ln1_g']), 'ln1b': row(p['edge_ln1_b']),
        'mw2': p['edge_w2'], 'mb2': row(p['edge_b2']),
        'ln2g': row(p['edge_ln2_g']), 'ln2b': row(p['edge_ln2_b']),
        'nw1a': p['node_w1'][:128], 'nw1b': p['node_w1'][128:],
        'nb1': row(p['node_b1']), 'nlng': row(p['node_ln_g']),
        'nlnb': row(p['node_ln_b']), 'nw2': p['node_w2'], 'nb2': row(p['node_b2']),
        'lng': row(p['ln_g']), 'lnb': row(p['ln_b']),
    }


def _egcl_layer(h, posn, row, col, row2d, col2d, zeros, wp, eb, npad):
    hr, hc, pr, pc = _sc_gather4(h, posn, row, col)
    df, db = _coord_pass(hr, hc, pr, pc, wp, eb)
    pos2 = _sc_scatter_pos(posn, df, db, row, col)
    pr2, pc2 = _sc_gather_pos2(pos2, row, col)
    m = _msg_pass(hr, hc, pr2, pc2, wp, eb)
    agg = _sc_scatter_msg(m, row2d, zeros)
    h2 = _node_pass(h, agg, wp)
    return h2, pos2


def kernel(x, pos, edge_index, batch, params):
    n, in_dim = x.shape
    e = edge_index.shape[1]
    g = 16
    eb = 2048
    npad = _rup(n + 1, 128)
    epad = _rup(e, 4096)
    nb = epad // eb

    xp = jnp.zeros((npad, 16), F32).at[:n, :in_dim].set(x)
    posp = jnp.zeros((npad, 128), F32).at[:n, :3].set(pos)
    row = jnp.full((epad,), n, jnp.int32).at[:e].set(edge_index[0])
    col = jnp.full((epad,), n, jnp.int32).at[:e].set(edge_index[1])
    row2d = row.reshape(epad // 128, 128)
    col2d = col.reshape(epad // 128, 128)
    zeros = jnp.zeros((npad, 128), F32)
    batchp = jnp.full((npad, 1), g, jnp.int32).at[:n, 0].set(batch)

    p = params
    projw = jnp.concatenate([p['proj_w'], jnp.zeros((16 - in_dim, 128), F32)], axis=0)
    w0 = _prep_egcl(p['egcl0'])
    w2 = _prep_egcl(p['egcl2'])
    fin = {
        'gw1': p['gate_w1'], 'gb1': p['gate_b1'].reshape(1, -1),
        'glg': p['gate_ln_g'].reshape(1, -1), 'glb': p['gate_ln_b'].reshape(1, -1),
        'gw2': p['gate_w2'], 'gb2': p['gate_b2'].reshape(1, -1),
        'gw3r': p['gate_w3'].reshape(1, -1), 'gb3': p['gate_b3'].reshape(1, 1),
        'ow1': p['out_w1'], 'ob1': p['out_b1'].reshape(1, -1),
        'olg': p['out_ln_g'].reshape(1, -1), 'olb': p['out_ln_b'].reshape(1, -1),
        'ow2': p['out_w2'], 'ob2': p['out_b2'].reshape(1, -1),
    }

    h, posn = _prep(xp, posp, projw, p['proj_b'].reshape(1, -1),
                    p['proj_ln_g'].reshape(1, -1), p['proj_ln_b'].reshape(1, -1), n)

    h, pos2 = _egcl_layer(h, posn, row, col, row2d, col2d, zeros, w0, eb, npad)
    posn = _center(pos2, n)

    agg = _sc_gc(h, row, col2d, zeros)
    h = _gc_pass(agg, h, p['gc1']['rel_w'], p['gc1']['rel_b'].reshape(1, -1),
                 p['gc1']['root_w'])

    h, pos2 = _egcl_layer(h, posn, row, col, row2d, col2d, zeros, w2, eb, npad)

    agg = _sc_gc(h, row, col2d, zeros)
    h = _gc_pass(agg, h, p['gc3']['rel_w'], p['gc3']['rel_b'].reshape(1, -1),
                 p['gc3']['root_w'])

    out_dim = p['out_w2'].shape[1]
    return _final_pass(h, batchp, fin, n, g, out_dim)


# speedup vs baseline: 1.0375x; 1.0375x over previous
import functools

import jax
import jax.numpy as jnp
from jax import lax
from jax.experimental import pallas as pl
from jax.experimental.pallas import tpu as pltpu
from jax.experimental.pallas import tpu_sc as plsc

F32 = jnp.float32
_CH = 128


def _sc_mesh():
    return plsc.VectorSubcoreMesh(core_axis_name="c", subcore_axis_name="s")


def _rup(x, m):
    return (x + m - 1) // m * m


def _silu(x):
    return x * (1.0 / (1.0 + jnp.exp(-x)))


def _sigmoid(x):
    return 1.0 / (1.0 + jnp.exp(-x))


def _gelu(x):
    return 0.5 * x * (1.0 + lax.erf(x * 0.7071067811865476))


def _ln(x, g, b, eps=1e-5):
    m = jnp.mean(x, axis=-1, keepdims=True)
    v = jnp.mean((x - m) ** 2, axis=-1, keepdims=True)
    return (x - m) * lax.rsqrt(v + eps) * g + b


def _dot(a, b):
    return jnp.dot(a, b, preferred_element_type=F32)


def _prep_body(n_real, x_ref, p_ref, w_ref, b_ref, g_ref, beta_ref, h_ref, pn_ref):
    npad = p_ref.shape[0]
    valid = (lax.broadcasted_iota(jnp.int32, (npad, 1), 0) < n_real).astype(F32)
    pos = p_ref[...]
    mean = jnp.sum(pos, axis=0, keepdims=True) * (1.0 / n_real)
    cen = (pos - mean) * valid
    rms = jnp.sqrt(jnp.sum(cen * cen) * (1.0 / n_real))
    pn_ref[...] = cen * (1.0 / jnp.maximum(rms, 1e-6))
    y = _dot(x_ref[...], w_ref[...]) + b_ref[...]
    h_ref[...] = _gelu(_ln(y, g_ref[...], beta_ref[...]))


def _prep(xp, posp, w, b, g, beta, n_real):
    npad = xp.shape[0]
    return pl.pallas_call(
        functools.partial(_prep_body, n_real),
        out_shape=(jax.ShapeDtypeStruct((npad, 128), F32),
                   jax.ShapeDtypeStruct((npad, 128), F32)),
    )(xp, posp, w, b, g, beta)


def _center_body(n_real, p_ref, o_ref):
    npad = p_ref.shape[0]
    valid = (lax.broadcasted_iota(jnp.int32, (npad, 1), 0) < n_real).astype(F32)
    pos = p_ref[...]
    mean = jnp.sum(pos * valid, axis=0, keepdims=True) * (1.0 / n_real)
    cen = (pos - mean) * valid
    rms = jnp.sqrt(jnp.sum(cen * cen) * (1.0 / n_real))
    o_ref[...] = cen * (1.0 / jnp.maximum(rms, 1e-6))


def _center(pos, n_real):
    return pl.pallas_call(
        functools.partial(_center_body, n_real),
        out_shape=jax.ShapeDtypeStruct(pos.shape, F32),
    )(pos)


def _gather4_body(h_ref, p_ref, r_ref, c_ref, hr_ref, hc_ref, pr_ref, pc_ref):
    bsz = hr_ref.shape[0]

    def body(i, _):
        r = r_ref[0, 0, i]
        c = c_ref[0, 0, i]
        hr_ref[pl.ds(i, 1), :] = h_ref[pl.ds(r, 1), :]
        hc_ref[pl.ds(i, 1), :] = h_ref[pl.ds(c, 1), :]
        pr_ref[pl.ds(i, 1), :] = p_ref[pl.ds(r, 1), :]
        pc_ref[pl.ds(i, 1), :] = p_ref[pl.ds(c, 1), :]
        return 0

    lax.fori_loop(0, bsz, body, 0)


def _gather4(h, pos, rowb, colb, eb):
    npad = h.shape[0]
    nb = rowb.shape[0]
    epad = nb * eb
    resident = lambda shp: pl.BlockSpec(shp, lambda i: (0, 0))
    idx_spec = pl.BlockSpec((1, 1, eb), lambda i: (i, 0, 0), memory_space=pltpu.SMEM)
    return pl.pallas_call(
        _gather4_body,
        grid=(nb,),
        in_specs=[resident((npad, 128)), resident((npad, 16)), idx_spec, idx_spec],
        out_specs=(pl.BlockSpec((eb, 128), lambda i: (i, 0)),
                   pl.BlockSpec((eb, 128), lambda i: (i, 0)),
                   pl.BlockSpec((eb, 16), lambda i: (i, 0)),
                   pl.BlockSpec((eb, 16), lambda i: (i, 0))),
        out_shape=(jax.ShapeDtypeStruct((epad, 128), F32),
                   jax.ShapeDtypeStruct((epad, 128), F32),
                   jax.ShapeDtypeStruct((epad, 16), F32),
                   jax.ShapeDtypeStruct((epad, 16), F32)),
    )(h, pos, rowb, colb)


def _gather2_body(p_ref, r_ref, c_ref, pr_ref, pc_ref):
    bsz = pr_ref.shape[0]

    def body(i, _):
        r = r_ref[0, 0, i]
        c = c_ref[0, 0, i]
        pr_ref[pl.ds(i, 1), :] = p_ref[pl.ds(r, 1), :]
        pc_ref[pl.ds(i, 1), :] = p_ref[pl.ds(c, 1), :]
        return 0

    lax.fori_loop(0, bsz, body, 0)


def _gather2(pos, rowb, colb, eb):
    npad = pos.shape[0]
    nb = rowb.shape[0]
    epad = nb * eb
    idx_spec = pl.BlockSpec((1, 1, eb), lambda i: (i, 0, 0), memory_space=pltpu.SMEM)
    return pl.pallas_call(
        _gather2_body,
        grid=(nb,),
        in_specs=[pl.BlockSpec((npad, 16), lambda i: (0, 0)), idx_spec, idx_spec],
        out_specs=(pl.BlockSpec((eb, 16), lambda i: (i, 0)),
                   pl.BlockSpec((eb, 16), lambda i: (i, 0))),
        out_shape=(jax.ShapeDtypeStruct((epad, 16), F32),
                   jax.ShapeDtypeStruct((epad, 16), F32)),
    )(pos, rowb, colb)


def _scat_pos_body(init_ref, df_ref, db_ref, r_ref, c_ref, out_ref):
    bsz = df_ref.shape[0]

    @pl.when(pl.program_id(0) == 0)
    def _():
        out_ref[...] = init_ref[...]

    def body(i, _):
        r = r_ref[0, 0, i]
        c = c_ref[0, 0, i]
        out_ref[pl.ds(r, 1), :] += df_ref[pl.ds(i, 1), :]
        out_ref[pl.ds(c, 1), :] += db_ref[pl.ds(i, 1), :]
        return 0

    lax.fori_loop(0, bsz, body, 0)


def _scatter_pos(init, df, db, rowb, colb, eb):
    npad = init.shape[0]
    nb = rowb.shape[0]
    idx_spec = pl.BlockSpec((1, 1, eb), lambda i: (i, 0, 0), memory_space=pltpu.SMEM)
    return pl.pallas_call(
        _scat_pos_body,
        grid=(nb,),
        in_specs=[pl.BlockSpec((npad, 16), lambda i: (0, 0)),
                  pl.BlockSpec((eb, 16), lambda i: (i, 0)),
                  pl.BlockSpec((eb, 16), lambda i: (i, 0)),
                  idx_spec, idx_spec],
        out_specs=pl.BlockSpec((npad, 16), lambda i: (0, 0)),
        out_shape=jax.ShapeDtypeStruct((npad, 16), F32),
    )(init, df, db, rowb, colb)


def _scat_msg_body(m_ref, r_ref, out_ref):
    bsz = m_ref.shape[0]

    @pl.when(pl.program_id(0) == 0)
    def _():
        out_ref[...] = jnp.zeros_like(out_ref)

    def body(i, _):
        r = r_ref[0, 0, i]
        out_ref[pl.ds(r, 1), :] += m_ref[pl.ds(i, 1), :]
        return 0

    lax.fori_loop(0, bsz, body, 0)


def _scatter_msg(m, rowb, npad, eb):
    nb = rowb.shape[0]
    idx_spec = pl.BlockSpec((1, 1, eb), lambda i: (i, 0, 0), memory_space=pltpu.SMEM)
    return pl.pallas_call(
        _scat_msg_body,
        grid=(nb,),
        in_specs=[pl.BlockSpec((eb, 128), lambda i: (i, 0)), idx_spec],
        out_specs=pl.BlockSpec((npad, 128), lambda i: (0, 0)),
        out_shape=jax.ShapeDtypeStruct((npad, 128), F32),
    )(m, rowb)


def _gc_gs_body(h_ref, r_ref, c_ref, out_ref):
    bsz = r_ref.shape[2]

    @pl.when(pl.program_id(0) == 0)
    def _():
        out_ref[...] = jnp.zeros_like(out_ref)

    def body(i, _):
        r = r_ref[0, 0, i]
        c = c_ref[0, 0, i]
        out_ref[pl.ds(c, 1), :] += h_ref[pl.ds(r, 1), :]
        return 0

    lax.fori_loop(0, bsz, body, 0)


def _gc_gather_scatter(h, rowb, colb, eb):
    npad = h.shape[0]
    nb = rowb.shape[0]
    idx_spec = pl.BlockSpec((1, 1, eb), lambda i: (i, 0, 0), memory_space=pltpu.SMEM)
    return pl.pallas_call(
        _gc_gs_body,
        grid=(nb,),
        in_specs=[pl.BlockSpec((npad, 128), lambda i: (0, 0)), idx_spec, idx_spec],
        out_specs=pl.BlockSpec((npad, 128), lambda i: (0, 0)),
        out_shape=jax.ShapeDtypeStruct((npad, 128), F32),
    )(h, rowb, colb)


def _sc_gather4(h, pos, row, col, zeros):
    npad = h.shape[0]
    epad = row.shape[0]
    ch = 64
    per_w = epad // 32
    nchunk = per_w // ch

    @functools.partial(
        pl.kernel,
        out_type=(jax.ShapeDtypeStruct((epad, 128), F32),
                  jax.ShapeDtypeStruct((epad, 128), F32),
                  jax.ShapeDtypeStruct((epad, 128), F32)),
        mesh=_sc_mesh(),
        scratch_types=[pltpu.VMEM((per_w,), jnp.int32), pltpu.VMEM((per_w,), jnp.int32),
                       pltpu.VMEM((2, ch, 128), F32), pltpu.VMEM((2, ch, 128), F32),
                       pltpu.VMEM((2, ch, 128), F32), pltpu.VMEM((2, ch, 128), F32),
                       pltpu.VMEM((2, ch, 128), F32),
                       pltpu.SemaphoreType.DMA, pltpu.SemaphoreType.DMA],
    )
    def k(h_hbm, p_hbm, r_hbm, c_hbm, z_hbm, hr_hbm, hc_hbm, rel_hbm,
          riv, civ, hrv, hcv, prv, pcv, relv, sg0, sg1):
        wid = lax.axis_index("s") * 2 + lax.axis_index("c")
        base = wid * per_w
        sems = (sg0, sg1)

        pltpu.sync_copy(r_hbm.at[pl.ds(base, per_w)], riv)
        pltpu.sync_copy(c_hbm.at[pl.ds(base, per_w)], civ)
        pltpu.sync_copy(z_hbm.at[pl.ds(0, ch)], relv.at[0])
        pltpu.sync_copy(z_hbm.at[pl.ds(0, ch)], relv.at[1])

        def fire(j, b):
            ri = riv.at[pl.ds(j * ch, ch)]
            ci = civ.at[pl.ds(j * ch, ch)]
            pltpu.async_copy(h_hbm.at[ri], hrv.at[b], sems[b])
            pltpu.async_copy(h_hbm.at[ci], hcv.at[b], sems[b])
            pltpu.async_copy(p_hbm.at[ri], prv.at[b], sems[b])
            pltpu.async_copy(p_hbm.at[ci], pcv.at[b], sems[b])

        def drain_write(j, b):
            ri = riv.at[pl.ds(j * ch, ch)]
            ci = civ.at[pl.ds(j * ch, ch)]
            pltpu.make_async_copy(h_hbm.at[ri], hrv.at[b], sems[b]).wait()
            pltpu.make_async_copy(h_hbm.at[ci], hcv.at[b], sems[b]).wait()
            pltpu.make_async_copy(p_hbm.at[ri], prv.at[b], sems[b]).wait()
            pltpu.make_async_copy(p_hbm.at[ci], pcv.at[b], sems[b]).wait()

            def rel_row(q, _):
                for u in range(4):
                    i = q * 4 + u
                    relv[b, i, pl.ds(0, 16)] = (prv[b, i, pl.ds(0, 16)] -
                                                pcv[b, i, pl.ds(0, 16)])
                return 0

            lax.fori_loop(0, ch // 4, rel_row, 0)
            off = base + j * ch
            pltpu.sync_copy(hrv.at[b], hr_hbm.at[pl.ds(off, ch)])
            pltpu.sync_copy(hcv.at[b], hc_hbm.at[pl.ds(off, ch)])
            pltpu.sync_copy(relv.at[b], rel_hbm.at[pl.ds(off, ch)])

        fire(0, 0)

        def body(st, _):
            j0 = 2 * st
            fire(j0 + 1, 1)
            drain_write(j0, 0)

            @pl.when(j0 + 2 < nchunk)
            def _():
                fire(j0 + 2, 0)

            drain_write(j0 + 1, 1)
            return 0

        lax.fori_loop(0, nchunk // 2, body, 0)

    return k(h, pos, row, col, zeros)


def _sc_gather_rel2(pos, row, col, zeros):
    epad = row.shape[0]
    ch = _CH
    per_w = epad // 32
    nchunk = per_w // ch

    @functools.partial(
        pl.kernel,
        out_type=jax.ShapeDtypeStruct((epad, 128), F32),
        mesh=_sc_mesh(),
        scratch_types=[pltpu.VMEM((per_w,), jnp.int32), pltpu.VMEM((per_w,), jnp.int32),
                       pltpu.VMEM((2, ch, 128), F32), pltpu.VMEM((2, ch, 128), F32),
                       pltpu.VMEM((2, ch, 128), F32),
                       pltpu.SemaphoreType.DMA, pltpu.SemaphoreType.DMA],
    )
    def k(p_hbm, r_hbm, c_hbm, z_hbm, rel_hbm, riv, civ, prv, pcv, relv, sg0, sg1):
        wid = lax.axis_index("s") * 2 + lax.axis_index("c")
        base = wid * per_w
        sems = (sg0, sg1)

        pltpu.sync_copy(r_hbm.at[pl.ds(base, per_w)], riv)
        pltpu.sync_copy(c_hbm.at[pl.ds(base, per_w)], civ)
        pltpu.sync_copy(z_hbm.at[pl.ds(0, ch)], relv.at[0])
        pltpu.sync_copy(z_hbm.at[pl.ds(0, ch)], relv.at[1])

        def fire(j, b):
            ri = riv.at[pl.ds(j * ch, ch)]
            ci = civ.at[pl.ds(j * ch, ch)]
            pltpu.async_copy(p_hbm.at[ri], prv.at[b], sems[b])
            pltpu.async_copy(p_hbm.at[ci], pcv.at[b], sems[b])

        def drain_write(j, b):
            ri = riv.at[pl.ds(j * ch, ch)]
            ci = civ.at[pl.ds(j * ch, ch)]
            pltpu.make_async_copy(p_hbm.at[ri], prv.at[b], sems[b]).wait()
            pltpu.make_async_copy(p_hbm.at[ci], pcv.at[b], sems[b]).wait()

            def rel_row(q, _):
                for u in range(4):
                    i = q * 4 + u
                    relv[b, i, pl.ds(0, 16)] = (prv[b, i, pl.ds(0, 16)] -
                                                pcv[b, i, pl.ds(0, 16)])
                return 0

            lax.fori_loop(0, ch // 4, rel_row, 0)
            off = base + j * ch
            pltpu.sync_copy(relv.at[b], rel_hbm.at[pl.ds(off, ch)])

        fire(0, 0)

        def body(st, _):
            j0 = 2 * st
            fire(j0 + 1, 1)
            drain_write(j0, 0)

            @pl.when(j0 + 2 < nchunk)
            def _():
                fire(j0 + 2, 0)

            drain_write(j0 + 1, 1)
            return 0

        lax.fori_loop(0, nchunk // 2, body, 0)

    return k(pos, row, col, zeros)


def _sc_scatter_pos(init, df, db, row, col):
    npad = init.shape[0]
    epad = row.shape[0]
    ch = _CH
    per_w = epad // 16
    nchunk = per_w // ch
    rps = npad // 16

    @functools.partial(
        pl.kernel,
        out_type=jax.ShapeDtypeStruct((npad, 128), F32),
        mesh=_sc_mesh(),
        scratch_types=[pltpu.VMEM((ch,), jnp.int32),
                       pltpu.VMEM((ch, 128), F32), pltpu.VMEM((ch, 128), F32),
                       pltpu.VMEM_SHARED((npad, 128), F32),
                       pltpu.SemaphoreType.DMA, pltpu.SemaphoreType.DMA],
    )
    def k(init_hbm, df_hbm, db_hbm, r_hbm, c_hbm, out_hbm,
          iv, vv0, vv1, acc, sg0, sg1):
        cid = lax.axis_index("c")
        sid = lax.axis_index("s")
        sems = (sg0, sg1)
        vvs = (vv0, vv1)

        @pl.when(cid == 0)
        def _():
            pltpu.sync_copy(init_hbm.at[pl.ds(sid * rps, rps)],
                            acc.at[pl.ds(sid * rps, rps)])
            plsc.subcore_barrier()
            base = sid * per_w

            def one_pass(v_hbm, i_hbm):
                def fire(j, b):
                    pltpu.async_copy(v_hbm.at[pl.ds(base + j * ch, ch)], vvs[b],
                                     sems[b])

                def drain_add(j, b):
                    pltpu.sync_copy(i_hbm.at[pl.ds(base + j * ch, ch)], iv)
                    pltpu.make_async_copy(v_hbm.at[pl.ds(base + j * ch, ch)],
                                          vvs[b], sems[b]).wait()
                    pltpu.sync_copy(vvs[b], acc.at[iv], add=True)

                fire(0, 0)

                def body(st, _):
                    j0 = 2 * st
                    fire(j0 + 1, 1)
                    drain_add(j0, 0)

                    @pl.when(j0 + 2 < nchunk)
                    def _():
                        fire(j0 + 2, 0)

                    drain_add(j0 + 1, 1)
                    return 0

                lax.fori_loop(0, nchunk // 2, body, 0)

            one_pass(df_hbm, r_hbm)
            one_pass(db_hbm, c_hbm)
            plsc.subcore_barrier()
            pltpu.sync_copy(acc.at[pl.ds(sid * rps, rps)],
                            out_hbm.at[pl.ds(sid * rps, rps)])

    return k(init, df, db, row, col)


def _sc_scatter_msg(m, row2d, zeros):
    npad = zeros.shape[0]
    ch = _CH
    nrow = row2d.shape[0]
    nchunk = nrow // 32
    per_w = nchunk * ch
    rps = npad // 16

    @functools.partial(
        pl.kernel,
        out_type=jax.ShapeDtypeStruct((2 * npad, 128), F32),
        mesh=_sc_mesh(),
        scratch_types=[pltpu.VMEM((nchunk, ch), jnp.int32),
                       pltpu.VMEM((ch, 128), F32), pltpu.VMEM((ch, 128), F32),
                       pltpu.VMEM_SHARED((npad, 128), F32),
                       pltpu.SemaphoreType.DMA, pltpu.SemaphoreType.DMA],
    )
    def k(m_hbm, r_hbm, z_hbm, out_hbm, riv, vv0, vv1, acc, sg0, sg1):
        cid = lax.axis_index("c")
        sid = lax.axis_index("s")
        sems = (sg0, sg1)
        vvs = (vv0, vv1)
        wid = sid * 2 + cid
        pltpu.sync_copy(z_hbm.at[pl.ds(sid * rps, rps)], acc.at[pl.ds(sid * rps, rps)])
        pltpu.sync_copy(r_hbm.at[pl.ds(wid * nchunk, nchunk)], riv)
        plsc.subcore_barrier()
        base = wid * per_w

        def fire(j, b):
            pltpu.async_copy(m_hbm.at[pl.ds(base + j * ch, ch)], vvs[b], sems[b])

        def drain_add(j, b):
            pltpu.make_async_copy(m_hbm.at[pl.ds(base + j * ch, ch)], vvs[b],
                                  sems[b]).wait()
            pltpu.sync_copy(vvs[b], acc.at[riv.at[j]], add=True)

        fire(0, 0)

        def body(st, _):
            j0 = 2 * st
            fire(j0 + 1, 1)
            drain_add(j0, 0)

            @pl.when(j0 + 2 < nchunk)
            def _():
                fire(j0 + 2, 0)

            drain_add(j0 + 1, 1)
            return 0

        lax.fori_loop(0, nchunk // 2, body, 0)
        plsc.subcore_barrier()
        pltpu.sync_copy(acc.at[pl.ds(sid * rps, rps)],
                        out_hbm.at[pl.ds(cid * npad + sid * rps, rps)])

    return k(m, row2d, zeros)


def _sc_gc(h, row, col2d, zeros):
    npad = h.shape[0]
    epad = row.shape[0]
    ch = _CH
    per_w = epad // 32
    nchunk = per_w // ch
    rps = npad // 16

    @functools.partial(
        pl.kernel,
        out_type=jax.ShapeDtypeStruct((2 * npad, 128), F32),
        mesh=_sc_mesh(),
        scratch_types=[pltpu.VMEM((per_w,), jnp.int32),
                       pltpu.VMEM((nchunk, ch), jnp.int32),
                       pltpu.VMEM((ch, 128), F32), pltpu.VMEM((ch, 128), F32),
                       pltpu.VMEM_SHARED((npad, 128), F32),
                       pltpu.SemaphoreType.DMA, pltpu.SemaphoreType.DMA],
    )
    def k(h_hbm, r_hbm, c_hbm, z_hbm, out_hbm, riv, civ, vv0, vv1, acc, sg0, sg1):
        cid = lax.axis_index("c")
        sid = lax.axis_index("s")
        sems = (sg0, sg1)
        vvs = (vv0, vv1)
        wid = sid * 2 + cid
        pltpu.sync_copy(z_hbm.at[pl.ds(sid * rps, rps)], acc.at[pl.ds(sid * rps, rps)])
        pltpu.sync_copy(r_hbm.at[pl.ds(wid * per_w, per_w)], riv)
        pltpu.sync_copy(c_hbm.at[pl.ds(wid * nchunk, nchunk)], civ)
        plsc.subcore_barrier()

        def fire(j, b):
            ri = riv.at[pl.ds(j * ch, ch)]
            pltpu.async_copy(h_hbm.at[ri], vvs[b], sems[b])

        def drain_add(j, b):
            ri = riv.at[pl.ds(j * ch, ch)]
            pltpu.make_async_copy(h_hbm.at[ri], vvs[b], sems[b]).wait()
            pltpu.sync_copy(vvs[b], acc.at[civ.at[j]], add=True)

        fire(0, 0)

        def body(st, _):
            j0 = 2 * st
            fire(j0 + 1, 1)
            drain_add(j0, 0)

            @pl.when(j0 + 2 < nchunk)
            def _():
                fire(j0 + 2, 0)

            drain_add(j0 + 1, 1)
            return 0

        lax.fori_loop(0, nchunk // 2, body, 0)
        plsc.subcore_barrier()
        pltpu.sync_copy(acc.at[pl.ds(sid * rps, rps)],
                        out_hbm.at[pl.ds(cid * npad + sid * rps, rps)])

    return k(h, row, col2d, zeros)


def _coord_body(hr_ref, hc_ref, rel_ref,
                w1a_ref, w1b_ref, w1c_ref, b1_ref, w2_ref, b2_ref,
                w3_ref, b3_ref, ew1_ref, ewb1_ref, ew2r_ref, ewb2_ref,
                scale_ref, df_ref, db_ref):
    hr = hr_ref[...]
    hc = hc_ref[...]
    rel = rel_ref[...]
    w1a = w1a_ref[...]
    w1b = w1b_ref[...]
    w1c = w1c_ref[...]
    b1 = b1_ref[...]
    ha = _dot(hr, w1a)
    hb = _dot(hc, w1b)
    hab = _dot(hc, w1a)
    hbb = _dot(hr, w1b)
    rc = _dot(rel, w1c)
    t1 = _silu(ha + hb + rc + b1)
    u1 = _silu(hab + hbb - rc + b1)
    t2 = _silu(_dot(t1, w2_ref[...]) + b2_ref[...])
    u2 = _silu(_dot(u1, w2_ref[...]) + b2_ref[...])
    raw_f = jnp.tanh(_dot(t2, w3_ref[...]) + b3_ref[...])
    raw_b = jnp.tanh(_dot(u2, w3_ref[...]) + b3_ref[...])
    edge_len = jnp.sqrt(jnp.sum(rel * rel, axis=-1, keepdims=True))
    s = jnp.clip(scale_ref[0, 0], 0.0, 5.0)
    nf = jnp.maximum(jnp.sqrt(jnp.sum(raw_f * raw_f, axis=-1, keepdims=True)), 1e-8)
    nb_ = jnp.maximum(jnp.sqrt(jnp.sum(raw_b * raw_b, axis=-1, keepdims=True)), 1e-8)
    ew1 = ew1_ref[...]
    ewb1 = ewb1_ref[...]
    ew2r = ew2r_ref[...]
    ewb2 = ewb2_ref[0, 0]
    ew_f = _sigmoid(jnp.sum(_silu(_dot(rel, ew1) + ewb1) * ew2r, axis=-1, keepdims=True) + ewb2)
    ew_b = _sigmoid(jnp.sum(_silu(_dot(-rel, ew1) + ewb1) * ew2r, axis=-1, keepdims=True) + ewb2)
    common = 0.05 * s * edge_len
    df_ref[...] = raw_f / nf * (common * ew_f)
    db_ref[...] = raw_b / nb_ * (common * ew_b)


def _coord_pass(hr, hc, rel, wp, eb):
    epad = hr.shape[0]
    nb = epad // eb
    ebspec = lambda d: pl.BlockSpec((eb, d), lambda i: (i, 0))
    wspec = lambda a: pl.BlockSpec(a.shape, lambda i: (0,) * a.ndim)
    weights = (wp['cw1a'], wp['cw1b'], wp['cw1c'], wp['cb1'], wp['cw2'], wp['cb2'],
               wp['cw3'], wp['cb3'], wp['ew1'], wp['ewb1'], wp['ew2r'], wp['ewb2'],
               wp['scale'])
    return pl.pallas_call(
        _coord_body,
        grid=(nb,),
        in_specs=[ebspec(128), ebspec(128), ebspec(128)] +
                 [wspec(a) for a in weights],
        out_specs=(ebspec(128), ebspec(128)),
        out_shape=(jax.ShapeDtypeStruct((epad, 128), F32),
                   jax.ShapeDtypeStruct((epad, 128), F32)),
    )(hr, hc, rel, *weights)


def _msg_body(hr_ref, hc_ref, rel_ref,
              w1a_ref, w1b_ref, w1c_ref, b1_ref, g1_ref, be1_ref,
              w2_ref, b2_ref, g2_ref, be2_ref, m_ref):
    rel = rel_ref[...]
    m1 = _silu(_dot(hr_ref[...], w1a_ref[...]) + _dot(hc_ref[...], w1b_ref[...]) +
               _dot(rel, w1c_ref[...]) + b1_ref[...])
    m1 = _ln(m1, g1_ref[...], be1_ref[...])
    m2 = _silu(_dot(m1, w2_ref[...]) + b2_ref[...])
    m_ref[...] = _ln(m2, g2_ref[...], be2_ref[...])


def _msg_pass(hr, hc, rel2, wp, eb):
    epad = hr.shape[0]
    nb = epad // eb
    ebspec = lambda d: pl.BlockSpec((eb, d), lambda i: (i, 0))
    wspec = lambda a: pl.BlockSpec(a.shape, lambda i: (0,) * a.ndim)
    weights = (wp['mw1a'], wp['mw1b'], wp['mw1c'], wp['mb1'], wp['ln1g'], wp['ln1b'],
               wp['mw2'], wp['mb2'], wp['ln2g'], wp['ln2b'])
    return pl.pallas_call(
        _msg_body,
        grid=(nb,),
        in_specs=[ebspec(128), ebspec(128), ebspec(128)] +
                 [wspec(a) for a in weights],
        out_specs=ebspec(128),
        out_shape=jax.ShapeDtypeStruct((epad, 128), F32),
    )(hr, hc, rel2, *weights)


def _node_body(h_ref, agg_ref, w1a_ref, w1b_ref, b1_ref, lg_ref, lb_ref,
               w2_ref, b2_ref, g_ref, be_ref, out_ref):
    h = h_ref[...]
    npad = h_ref.shape[0]
    agg = agg_ref[pl.ds(0, npad), :] + agg_ref[pl.ds(npad, npad), :]
    nm = _silu(_dot(h, w1a_ref[...]) + _dot(agg, w1b_ref[...]) + b1_ref[...])
    nm = _ln(nm, lg_ref[...], lb_ref[...])
    nm = _dot(nm, w2_ref[...]) + b2_ref[...]
    out_ref[...] = _ln(h + nm, g_ref[...], be_ref[...])


def _node_pass(h, agg, wp):
    npad = h.shape[0]
    weights = (wp['nw1a'], wp['nw1b'], wp['nb1'], wp['nlng'], wp['nlnb'],
               wp['nw2'], wp['nb2'], wp['lng'], wp['lnb'])
    return pl.pallas_call(
        _node_body,
        out_shape=jax.ShapeDtypeStruct((npad, 128), F32),
    )(h, agg, *weights)


def _gc_body(agg_ref, h_ref, rw_ref, rb_ref, rootw_ref, out_ref):
    npad = h_ref.shape[0]
    agg = agg_ref[pl.ds(0, npad), :] + agg_ref[pl.ds(npad, npad), :]
    out_ref[...] = (_dot(agg, rw_ref[...]) + rb_ref[...] +
                    _dot(h_ref[...], rootw_ref[...]))


def _gc_pass(agg, h, rw, rb, rootw):
    npad = h.shape[0]
    return pl.pallas_call(
        _gc_body,
        out_shape=jax.ShapeDtypeStruct((npad, 128), F32),
    )(agg, h, rw, rb, rootw)


def _final_body(n_real, n_groups, h_ref, batch_ref,
                gw1_ref, gb1_ref, glg_ref, glb_ref, gw2_ref, gb2_ref,
                gw3r_ref, gb3_ref, ow1_ref, ob1_ref, olg_ref, olb_ref,
                ow2_ref, ob2_ref, out_ref):
    npad = h_ref.shape[0]
    h = h_ref[...]
    gate = _dot(h, gw1_ref[...]) + gb1_ref[...]
    gate = _gelu(_ln(gate, glg_ref[...], glb_ref[...]))
    gate = _gelu(_dot(gate, gw2_ref[...]) + gb2_ref[...])
    gate_s = jnp.sum(gate * gw3r_ref[...], axis=-1, keepdims=True) + gb3_ref[0, 0]
    valid = lax.broadcasted_iota(jnp.int32, (npad, 1), 0) < n_real
    gid = lax.broadcasted_iota(jnp.int32, (1, n_groups), 1)
    oh = jnp.logical_and(batch_ref[...] == gid, valid).astype(F32)
    gmax = jnp.max(jnp.where(oh > 0, gate_s, -1e30), axis=0, keepdims=True)
    gmax_g = jnp.sum(oh * gmax, axis=-1, keepdims=True)
    ex = jnp.where(valid, jnp.exp(gate_s - gmax_g), 0.0)
    den = jnp.sum(oh * ex, axis=0, keepdims=True)
    den_g = jnp.sum(oh * den, axis=-1, keepdims=True)
    attn = ex / jnp.maximum(den_g, 1e-16)
    pooled = lax.dot_general(oh, attn * h, (((0,), (0,)), ((), ())),
                             preferred_element_type=F32)
    o = _gelu(_ln(_dot(pooled, ow1_ref[...]) + ob1_ref[...], olg_ref[...], olb_ref[...]))
    out_ref[...] = _dot(o, ow2_ref[...]) + ob2_ref[...]


def _final_pass(h, batchp, wp, n_real, n_groups, out_dim):
    weights = (wp['gw1'], wp['gb1'], wp['glg'], wp['glb'], wp['gw2'], wp['gb2'],
               wp['gw3r'], wp['gb3'], wp['ow1'], wp['ob1'], wp['olg'], wp['olb'],
               wp['ow2'], wp['ob2'])
    return pl.pallas_call(
        functools.partial(_final_body, n_real, n_groups),
        out_shape=jax.ShapeDtypeStruct((n_groups, out_dim), F32),
    )(h, batchp, *weights)


def _prep_egcl(p):
    pad_rows = lambda w, r: jnp.concatenate(
        [w, jnp.zeros((r - w.shape[0], w.shape[1]), F32)], axis=0)
    pad_cols = lambda w, c: jnp.concatenate(
        [w, jnp.zeros((w.shape[0], c - w.shape[1]), F32)], axis=1)
    row = lambda v: v.reshape(1, -1)
    return {
        'cw1a': p['coord_w1'][:128], 'cw1b': p['coord_w1'][128:256],
        'cw1c': pad_rows(p['coord_w1'][256:], 128), 'cb1': row(p['coord_b1']),
        'cw2': p['coord_w2'], 'cb2': row(p['coord_b2']),
        'cw3': pad_cols(p['coord_w3'], 128), 'cb3': pad_cols(row(p['coord_b3']), 128),
        'ew1': pad_rows(p['ew_w1'], 128), 'ewb1': row(p['ew_b1']),
        'ew2r': p['ew_w2'].reshape(1, -1), 'ewb2': p['ew_b2'].reshape(1, 1),
        'scale': p['scale'].reshape(1, 1),
        'mw1a': p['edge_w1'][:128], 'mw1b': p['edge_w1'][128:256],
        'mw1c': pad_rows(p['edge_w1'][256:], 128), 'mb1': row(p['edge_b1']),
        'ln1g': row(p['edge_ln1_g']), 'ln1b': row(p['edge_ln1_b']),
        'mw2': p['edge_w2'], 'mb2': row(p['edge_b2']),
        'ln2g': row(p['edge_ln2_g']), 'ln2b': row(p['edge_ln2_b']),
        'nw1a': p['node_w1'][:128], 'nw1b': p['node_w1'][128:],
        'nb1': row(p['node_b1']), 'nlng': row(p['node_ln_g']),
        'nlnb': row(p['node_ln_b']), 'nw2': p['node_w2'], 'nb2': row(p['node_b2']),
        'lng': row(p['ln_g']), 'lnb': row(p['ln_b']),
    }


def _egcl_layer(h, posn, row, col, row2d, col2d, zeros, wp, eb, npad):
    hr, hc, rel = _sc_gather4(h, posn, row, col, zeros)
    df, db = _coord_pass(hr, hc, rel, wp, eb)
    pos2 = _sc_scatter_pos(posn, df, db, row, col)
    rel2 = _sc_gather_rel2(pos2, row, col, zeros)
    m = _msg_pass(hr, hc, rel2, wp, eb)
    agg = _sc_scatter_msg(m, row2d, zeros)
    h2 = _node_pass(h, agg, wp)
    return h2, pos2


def kernel(x, pos, edge_index, batch, params):
    n, in_dim = x.shape
    e = edge_index.shape[1]
    g = 16
    eb = 2048
    npad = _rup(n + 1, 128)
    epad = _rup(e, 4096)
    nb = epad // eb

    xp = jnp.zeros((npad, 16), F32).at[:n, :in_dim].set(x)
    posp = jnp.zeros((npad, 128), F32).at[:n, :3].set(pos)
    row = jnp.full((epad,), n, jnp.int32).at[:e].set(edge_index[0])
    col = jnp.full((epad,), n, jnp.int32).at[:e].set(edge_index[1])
    row2d = row.reshape(epad // 128, 128)
    col2d = col.reshape(epad // 128, 128)
    zeros = jnp.zeros((npad, 128), F32)
    batchp = jnp.full((npad, 1), g, jnp.int32).at[:n, 0].set(batch)

    p = params
    projw = jnp.concatenate([p['proj_w'], jnp.zeros((16 - in_dim, 128), F32)], axis=0)
    w0 = _prep_egcl(p['egcl0'])
    w2 = _prep_egcl(p['egcl2'])
    fin = {
        'gw1': p['gate_w1'], 'gb1': p['gate_b1'].reshape(1, -1),
        'glg': p['gate_ln_g'].reshape(1, -1), 'glb': p['gate_ln_b'].reshape(1, -1),
        'gw2': p['gate_w2'], 'gb2': p['gate_b2'].reshape(1, -1),
        'gw3r': p['gate_w3'].reshape(1, -1), 'gb3': p['gate_b3'].reshape(1, 1),
        'ow1': p['out_w1'], 'ob1': p['out_b1'].reshape(1, -1),
        'olg': p['out_ln_g'].reshape(1, -1), 'olb': p['out_ln_b'].reshape(1, -1),
        'ow2': p['out_w2'], 'ob2': p['out_b2'].reshape(1, -1),
    }

    h, posn = _prep(xp, posp, projw, p['proj_b'].reshape(1, -1),
                    p['proj_ln_g'].reshape(1, -1), p['proj_ln_b'].reshape(1, -1), n)

    h, pos2 = _egcl_layer(h, posn, row, col, row2d, col2d, zeros, w0, eb, npad)
    posn = _center(pos2, n)

    agg = _sc_gc(h, row, col2d, zeros)
    h = _gc_pass(agg, h, p['gc1']['rel_w'], p['gc1']['rel_b'].reshape(1, -1),
                 p['gc1']['root_w'])

    h, pos2 = _egcl_layer(h, posn, row, col, row2d, col2d, zeros, w2, eb, npad)

    agg = _sc_gc(h, row, col2d, zeros)
    h = _gc_pass(agg, h, p['gc3']['rel_w'], p['gc3']['rel_b'].reshape(1, -1),
                 p['gc3']['root_w'])

    out_dim = p['out_w2'].shape[1]
    return _final_pass(h, batchp, fin, n, g, out_dim)


# speedup vs baseline: 1.0387x; 1.0012x over previous
import functools

import jax
import jax.numpy as jnp
from jax import lax
from jax.experimental import pallas as pl
from jax.experimental.pallas import tpu as pltpu
from jax.experimental.pallas import tpu_sc as plsc

F32 = jnp.float32
_CH = 128


def _sc_mesh():
    return plsc.VectorSubcoreMesh(core_axis_name="c", subcore_axis_name="s")


def _rup(x, m):
    return (x + m - 1) // m * m


def _silu(x):
    return x * (1.0 / (1.0 + jnp.exp(-x)))


def _sigmoid(x):
    return 1.0 / (1.0 + jnp.exp(-x))


def _gelu(x):
    return 0.5 * x * (1.0 + lax.erf(x * 0.7071067811865476))


def _ln(x, g, b, eps=1e-5):
    m = jnp.mean(x, axis=-1, keepdims=True)
    v = jnp.mean((x - m) ** 2, axis=-1, keepdims=True)
    return (x - m) * lax.rsqrt(v + eps) * g + b


def _dot(a, b):
    return jnp.dot(a, b, preferred_element_type=F32)


def _prep_body(n_real, x_ref, p_ref, w_ref, b_ref, g_ref, beta_ref, h_ref, pn_ref):
    npad = p_ref.shape[0]
    valid = (lax.broadcasted_iota(jnp.int32, (npad, 1), 0) < n_real).astype(F32)
    pos = p_ref[...]
    mean = jnp.sum(pos, axis=0, keepdims=True) * (1.0 / n_real)
    cen = (pos - mean) * valid
    rms = jnp.sqrt(jnp.sum(cen * cen) * (1.0 / n_real))
    pn_ref[...] = cen * (1.0 / jnp.maximum(rms, 1e-6))
    y = _dot(x_ref[...], w_ref[...]) + b_ref[...]
    h_ref[...] = _gelu(_ln(y, g_ref[...], beta_ref[...]))


def _prep(xp, posp, w, b, g, beta, n_real):
    npad = xp.shape[0]
    return pl.pallas_call(
        functools.partial(_prep_body, n_real),
        out_shape=(jax.ShapeDtypeStruct((npad, 128), F32),
                   jax.ShapeDtypeStruct((npad, 128), F32)),
    )(xp, posp, w, b, g, beta)


def _center_body(n_real, p_ref, o_ref):
    npad = p_ref.shape[0]
    valid = (lax.broadcasted_iota(jnp.int32, (npad, 1), 0) < n_real).astype(F32)
    pos = p_ref[...]
    mean = jnp.sum(pos * valid, axis=0, keepdims=True) * (1.0 / n_real)
    cen = (pos - mean) * valid
    rms = jnp.sqrt(jnp.sum(cen * cen) * (1.0 / n_real))
    o_ref[...] = cen * (1.0 / jnp.maximum(rms, 1e-6))


def _center(pos, n_real):
    return pl.pallas_call(
        functools.partial(_center_body, n_real),
        out_shape=jax.ShapeDtypeStruct(pos.shape, F32),
    )(pos)


def _gather4_body(h_ref, p_ref, r_ref, c_ref, hr_ref, hc_ref, pr_ref, pc_ref):
    bsz = hr_ref.shape[0]

    def body(i, _):
        r = r_ref[0, 0, i]
        c = c_ref[0, 0, i]
        hr_ref[pl.ds(i, 1), :] = h_ref[pl.ds(r, 1), :]
        hc_ref[pl.ds(i, 1), :] = h_ref[pl.ds(c, 1), :]
        pr_ref[pl.ds(i, 1), :] = p_ref[pl.ds(r, 1), :]
        pc_ref[pl.ds(i, 1), :] = p_ref[pl.ds(c, 1), :]
        return 0

    lax.fori_loop(0, bsz, body, 0)


def _gather4(h, pos, rowb, colb, eb):
    npad = h.shape[0]
    nb = rowb.shape[0]
    epad = nb * eb
    resident = lambda shp: pl.BlockSpec(shp, lambda i: (0, 0))
    idx_spec = pl.BlockSpec((1, 1, eb), lambda i: (i, 0, 0), memory_space=pltpu.SMEM)
    return pl.pallas_call(
        _gather4_body,
        grid=(nb,),
        in_specs=[resident((npad, 128)), resident((npad, 16)), idx_spec, idx_spec],
        out_specs=(pl.BlockSpec((eb, 128), lambda i: (i, 0)),
                   pl.BlockSpec((eb, 128), lambda i: (i, 0)),
                   pl.BlockSpec((eb, 16), lambda i: (i, 0)),
                   pl.BlockSpec((eb, 16), lambda i: (i, 0))),
        out_shape=(jax.ShapeDtypeStruct((epad, 128), F32),
                   jax.ShapeDtypeStruct((epad, 128), F32),
                   jax.ShapeDtypeStruct((epad, 16), F32),
                   jax.ShapeDtypeStruct((epad, 16), F32)),
    )(h, pos, rowb, colb)


def _gather2_body(p_ref, r_ref, c_ref, pr_ref, pc_ref):
    bsz = pr_ref.shape[0]

    def body(i, _):
        r = r_ref[0, 0, i]
        c = c_ref[0, 0, i]
        pr_ref[pl.ds(i, 1), :] = p_ref[pl.ds(r, 1), :]
        pc_ref[pl.ds(i, 1), :] = p_ref[pl.ds(c, 1), :]
        return 0

    lax.fori_loop(0, bsz, body, 0)


def _gather2(pos, rowb, colb, eb):
    npad = pos.shape[0]
    nb = rowb.shape[0]
    epad = nb * eb
    idx_spec = pl.BlockSpec((1, 1, eb), lambda i: (i, 0, 0), memory_space=pltpu.SMEM)
    return pl.pallas_call(
        _gather2_body,
        grid=(nb,),
        in_specs=[pl.BlockSpec((npad, 16), lambda i: (0, 0)), idx_spec, idx_spec],
        out_specs=(pl.BlockSpec((eb, 16), lambda i: (i, 0)),
                   pl.BlockSpec((eb, 16), lambda i: (i, 0))),
        out_shape=(jax.ShapeDtypeStruct((epad, 16), F32),
                   jax.ShapeDtypeStruct((epad, 16), F32)),
    )(pos, rowb, colb)


def _scat_pos_body(init_ref, df_ref, db_ref, r_ref, c_ref, out_ref):
    bsz = df_ref.shape[0]

    @pl.when(pl.program_id(0) == 0)
    def _():
        out_ref[...] = init_ref[...]

    def body(i, _):
        r = r_ref[0, 0, i]
        c = c_ref[0, 0, i]
        out_ref[pl.ds(r, 1), :] += df_ref[pl.ds(i, 1), :]
        out_ref[pl.ds(c, 1), :] += db_ref[pl.ds(i, 1), :]
        return 0

    lax.fori_loop(0, bsz, body, 0)


def _scatter_pos(init, df, db, rowb, colb, eb):
    npad = init.shape[0]
    nb = rowb.shape[0]
    idx_spec = pl.BlockSpec((1, 1, eb), lambda i: (i, 0, 0), memory_space=pltpu.SMEM)
    return pl.pallas_call(
        _scat_pos_body,
        grid=(nb,),
        in_specs=[pl.BlockSpec((npad, 16), lambda i: (0, 0)),
                  pl.BlockSpec((eb, 16), lambda i: (i, 0)),
                  pl.BlockSpec((eb, 16), lambda i: (i, 0)),
                  idx_spec, idx_spec],
        out_specs=pl.BlockSpec((npad, 16), lambda i: (0, 0)),
        out_shape=jax.ShapeDtypeStruct((npad, 16), F32),
    )(init, df, db, rowb, colb)


def _scat_msg_body(m_ref, r_ref, out_ref):
    bsz = m_ref.shape[0]

    @pl.when(pl.program_id(0) == 0)
    def _():
        out_ref[...] = jnp.zeros_like(out_ref)

    def body(i, _):
        r = r_ref[0, 0, i]
        out_ref[pl.ds(r, 1), :] += m_ref[pl.ds(i, 1), :]
        return 0

    lax.fori_loop(0, bsz, body, 0)


def _scatter_msg(m, rowb, npad, eb):
    nb = rowb.shape[0]
    idx_spec = pl.BlockSpec((1, 1, eb), lambda i: (i, 0, 0), memory_space=pltpu.SMEM)
    return pl.pallas_call(
        _scat_msg_body,
        grid=(nb,),
        in_specs=[pl.BlockSpec((eb, 128), lambda i: (i, 0)), idx_spec],
        out_specs=pl.BlockSpec((npad, 128), lambda i: (0, 0)),
        out_shape=jax.ShapeDtypeStruct((npad, 128), F32),
    )(m, rowb)


def _gc_gs_body(h_ref, r_ref, c_ref, out_ref):
    bsz = r_ref.shape[2]

    @pl.when(pl.program_id(0) == 0)
    def _():
        out_ref[...] = jnp.zeros_like(out_ref)

    def body(i, _):
        r = r_ref[0, 0, i]
        c = c_ref[0, 0, i]
        out_ref[pl.ds(c, 1), :] += h_ref[pl.ds(r, 1), :]
        return 0

    lax.fori_loop(0, bsz, body, 0)


def _gc_gather_scatter(h, rowb, colb, eb):
    npad = h.shape[0]
    nb = rowb.shape[0]
    idx_spec = pl.BlockSpec((1, 1, eb), lambda i: (i, 0, 0), memory_space=pltpu.SMEM)
    return pl.pallas_call(
        _gc_gs_body,
        grid=(nb,),
        in_specs=[pl.BlockSpec((npad, 128), lambda i: (0, 0)), idx_spec, idx_spec],
        out_specs=pl.BlockSpec((npad, 128), lambda i: (0, 0)),
        out_shape=jax.ShapeDtypeStruct((npad, 128), F32),
    )(h, rowb, colb)


def _sc_gather4(h, pos, row, col, zeros):
    npad = h.shape[0]
    epad = row.shape[0]
    ch = 64
    per_w = epad // 32
    nchunk = per_w // ch
    assert nchunk % 3 == 2

    @functools.partial(
        pl.kernel,
        out_type=(jax.ShapeDtypeStruct((epad, 128), F32),
                  jax.ShapeDtypeStruct((epad, 128), F32),
                  jax.ShapeDtypeStruct((epad, 128), F32)),
        mesh=_sc_mesh(),
        scratch_types=[pltpu.VMEM((per_w,), jnp.int32), pltpu.VMEM((per_w,), jnp.int32),
                       pltpu.VMEM((3, ch, 128), F32), pltpu.VMEM((3, ch, 128), F32),
                       pltpu.VMEM((3, ch, 128), F32), pltpu.VMEM((3, ch, 128), F32),
                       pltpu.VMEM((ch, 128), F32),
                       pltpu.SemaphoreType.DMA, pltpu.SemaphoreType.DMA,
                       pltpu.SemaphoreType.DMA],
    )
    def k(h_hbm, p_hbm, r_hbm, c_hbm, z_hbm, hr_hbm, hc_hbm, rel_hbm,
          riv, civ, hrv, hcv, prv, pcv, relv, sg0, sg1, sg2):
        wid = lax.axis_index("s") * 2 + lax.axis_index("c")
        base = wid * per_w
        sems = (sg0, sg1, sg2)

        pltpu.sync_copy(r_hbm.at[pl.ds(base, per_w)], riv)
        pltpu.sync_copy(c_hbm.at[pl.ds(base, per_w)], civ)
        pltpu.sync_copy(z_hbm.at[pl.ds(0, ch)], relv)

        def fire(j, b):
            ri = riv.at[pl.ds(j * ch, ch)]
            ci = civ.at[pl.ds(j * ch, ch)]
            pltpu.async_copy(h_hbm.at[ri], hrv.at[b], sems[b])
            pltpu.async_copy(h_hbm.at[ci], hcv.at[b], sems[b])
            pltpu.async_copy(p_hbm.at[ri], prv.at[b], sems[b])
            pltpu.async_copy(p_hbm.at[ci], pcv.at[b], sems[b])

        def drain_write(j, b):
            ri = riv.at[pl.ds(j * ch, ch)]
            ci = civ.at[pl.ds(j * ch, ch)]
            pltpu.make_async_copy(h_hbm.at[ri], hrv.at[b], sems[b]).wait()
            pltpu.make_async_copy(h_hbm.at[ci], hcv.at[b], sems[b]).wait()
            pltpu.make_async_copy(p_hbm.at[ri], prv.at[b], sems[b]).wait()
            pltpu.make_async_copy(p_hbm.at[ci], pcv.at[b], sems[b]).wait()

            def rel_row(q, _):
                for u in range(4):
                    i = q * 4 + u
                    relv[i, pl.ds(0, 16)] = (prv[b, i, pl.ds(0, 16)] -
                                             pcv[b, i, pl.ds(0, 16)])
                return 0

            lax.fori_loop(0, ch // 4, rel_row, 0)
            off = base + j * ch
            pltpu.sync_copy(hrv.at[b], hr_hbm.at[pl.ds(off, ch)])
            pltpu.sync_copy(hcv.at[b], hc_hbm.at[pl.ds(off, ch)])
            pltpu.sync_copy(relv, rel_hbm.at[pl.ds(off, ch)])

        fire(0, 0)
        fire(1, 1)

        def body(st, _):
            j0 = 3 * st
            fire(j0 + 2, 2)
            drain_write(j0, 0)
            fire(j0 + 3, 0)
            drain_write(j0 + 1, 1)
            fire(j0 + 4, 1)
            drain_write(j0 + 2, 2)
            return 0

        lax.fori_loop(0, (nchunk - 2) // 3, body, 0)
        drain_write(nchunk - 2, 0)
        drain_write(nchunk - 1, 1)

    return k(h, pos, row, col, zeros)


def _sc_gather_rel2(pos, row, col, zeros):
    epad = row.shape[0]
    ch = 64
    per_w = epad // 32
    nchunk = per_w // ch
    assert nchunk % 3 == 2

    @functools.partial(
        pl.kernel,
        out_type=jax.ShapeDtypeStruct((epad, 128), F32),
        mesh=_sc_mesh(),
        scratch_types=[pltpu.VMEM((per_w,), jnp.int32), pltpu.VMEM((per_w,), jnp.int32),
                       pltpu.VMEM((3, ch, 128), F32), pltpu.VMEM((3, ch, 128), F32),
                       pltpu.VMEM((ch, 128), F32),
                       pltpu.SemaphoreType.DMA, pltpu.SemaphoreType.DMA,
                       pltpu.SemaphoreType.DMA],
    )
    def k(p_hbm, r_hbm, c_hbm, z_hbm, rel_hbm, riv, civ, prv, pcv, relv,
          sg0, sg1, sg2):
        wid = lax.axis_index("s") * 2 + lax.axis_index("c")
        base = wid * per_w
        sems = (sg0, sg1, sg2)

        pltpu.sync_copy(r_hbm.at[pl.ds(base, per_w)], riv)
        pltpu.sync_copy(c_hbm.at[pl.ds(base, per_w)], civ)
        pltpu.sync_copy(z_hbm.at[pl.ds(0, ch)], relv)

        def fire(j, b):
            ri = riv.at[pl.ds(j * ch, ch)]
            ci = civ.at[pl.ds(j * ch, ch)]
            pltpu.async_copy(p_hbm.at[ri], prv.at[b], sems[b])
            pltpu.async_copy(p_hbm.at[ci], pcv.at[b], sems[b])

        def drain_write(j, b):
            ri = riv.at[pl.ds(j * ch, ch)]
            ci = civ.at[pl.ds(j * ch, ch)]
            pltpu.make_async_copy(p_hbm.at[ri], prv.at[b], sems[b]).wait()
            pltpu.make_async_copy(p_hbm.at[ci], pcv.at[b], sems[b]).wait()

            def rel_row(q, _):
                for u in range(4):
                    i = q * 4 + u
                    relv[i, pl.ds(0, 16)] = (prv[b, i, pl.ds(0, 16)] -
                                             pcv[b, i, pl.ds(0, 16)])
                return 0

            lax.fori_loop(0, ch // 4, rel_row, 0)
            off = base + j * ch
            pltpu.sync_copy(relv, rel_hbm.at[pl.ds(off, ch)])

        fire(0, 0)
        fire(1, 1)

        def body(st, _):
            j0 = 3 * st
            fire(j0 + 2, 2)
            drain_write(j0, 0)
            fire(j0 + 3, 0)
            drain_write(j0 + 1, 1)
            fire(j0 + 4, 1)
            drain_write(j0 + 2, 2)
            return 0

        lax.fori_loop(0, (nchunk - 2) // 3, body, 0)
        drain_write(nchunk - 2, 0)
        drain_write(nchunk - 1, 1)

    return k(pos, row, col, zeros)


def _sc_scatter_pos(init, df, db, row, col):
    npad = init.shape[0]
    epad = row.shape[0]
    ch = _CH
    per_w = epad // 16
    nchunk = per_w // ch
    rps = npad // 16

    @functools.partial(
        pl.kernel,
        out_type=jax.ShapeDtypeStruct((npad, 128), F32),
        mesh=_sc_mesh(),
        scratch_types=[pltpu.VMEM((ch,), jnp.int32),
                       pltpu.VMEM((ch, 128), F32), pltpu.VMEM((ch, 128), F32),
                       pltpu.VMEM_SHARED((npad, 128), F32),
                       pltpu.SemaphoreType.DMA, pltpu.SemaphoreType.DMA],
    )
    def k(init_hbm, df_hbm, db_hbm, r_hbm, c_hbm, out_hbm,
          iv, vv0, vv1, acc, sg0, sg1):
        cid = lax.axis_index("c")
        sid = lax.axis_index("s")
        sems = (sg0, sg1)
        vvs = (vv0, vv1)

        @pl.when(cid == 0)
        def _():
            pltpu.sync_copy(init_hbm.at[pl.ds(sid * rps, rps)],
                            acc.at[pl.ds(sid * rps, rps)])
            plsc.subcore_barrier()
            base = sid * per_w

            def one_pass(v_hbm, i_hbm):
                def fire(j, b):
                    pltpu.async_copy(v_hbm.at[pl.ds(base + j * ch, ch)], vvs[b],
                                     sems[b])

                def drain_add(j, b):
                    pltpu.sync_copy(i_hbm.at[pl.ds(base + j * ch, ch)], iv)
                    pltpu.make_async_copy(v_hbm.at[pl.ds(base + j * ch, ch)],
                                          vvs[b], sems[b]).wait()
                    pltpu.sync_copy(vvs[b], acc.at[iv], add=True)

                fire(0, 0)

                def body(st, _):
                    j0 = 2 * st
                    fire(j0 + 1, 1)
                    drain_add(j0, 0)

                    @pl.when(j0 + 2 < nchunk)
                    def _():
                        fire(j0 + 2, 0)

                    drain_add(j0 + 1, 1)
                    return 0

                lax.fori_loop(0, nchunk // 2, body, 0)

            one_pass(df_hbm, r_hbm)
            one_pass(db_hbm, c_hbm)
            plsc.subcore_barrier()
            pltpu.sync_copy(acc.at[pl.ds(sid * rps, rps)],
                            out_hbm.at[pl.ds(sid * rps, rps)])

    return k(init, df, db, row, col)


def _sc_scatter_msg(m, row2d, zeros):
    npad = zeros.shape[0]
    ch = _CH
    nrow = row2d.shape[0]
    nchunk = nrow // 32
    per_w = nchunk * ch
    rps = npad // 16

    @functools.partial(
        pl.kernel,
        out_type=jax.ShapeDtypeStruct((2 * npad, 128), F32),
        mesh=_sc_mesh(),
        scratch_types=[pltpu.VMEM((nchunk, ch), jnp.int32),
                       pltpu.VMEM((ch, 128), F32), pltpu.VMEM((ch, 128), F32),
                       pltpu.VMEM_SHARED((npad, 128), F32),
                       pltpu.SemaphoreType.DMA, pltpu.SemaphoreType.DMA],
    )
    def k(m_hbm, r_hbm, z_hbm, out_hbm, riv, vv0, vv1, acc, sg0, sg1):
        cid = lax.axis_index("c")
        sid = lax.axis_index("s")
        sems = (sg0, sg1)
        vvs = (vv0, vv1)
        wid = sid * 2 + cid
        pltpu.sync_copy(z_hbm.at[pl.ds(sid * rps, rps)], acc.at[pl.ds(sid * rps, rps)])
        pltpu.sync_copy(r_hbm.at[pl.ds(wid * nchunk, nchunk)], riv)
        plsc.subcore_barrier()
        base = wid * per_w

        def fire(j, b):
            pltpu.async_copy(m_hbm.at[pl.ds(base + j * ch, ch)], vvs[b], sems[b])

        def drain_add(j, b):
            pltpu.make_async_copy(m_hbm.at[pl.ds(base + j * ch, ch)], vvs[b],
                                  sems[b]).wait()
            pltpu.sync_copy(vvs[b], acc.at[riv.at[j]], add=True)

        fire(0, 0)

        def body(st, _):
            j0 = 2 * st
            fire(j0 + 1, 1)
            drain_add(j0, 0)

            @pl.when(j0 + 2 < nchunk)
            def _():
                fire(j0 + 2, 0)

            drain_add(j0 + 1, 1)
            return 0

        lax.fori_loop(0, nchunk // 2, body, 0)
        plsc.subcore_barrier()
        pltpu.sync_copy(acc.at[pl.ds(sid * rps, rps)],
                        out_hbm.at[pl.ds(cid * npad + sid * rps, rps)])

    return k(m, row2d, zeros)


def _sc_gc(h, row, col2d, zeros):
    npad = h.shape[0]
    epad = row.shape[0]
    ch = _CH
    per_w = epad // 32
    nchunk = per_w // ch
    rps = npad // 16

    @functools.partial(
        pl.kernel,
        out_type=jax.ShapeDtypeStruct((2 * npad, 128), F32),
        mesh=_sc_mesh(),
        scratch_types=[pltpu.VMEM((per_w,), jnp.int32),
                       pltpu.VMEM((nchunk, ch), jnp.int32),
                       pltpu.VMEM((ch, 128), F32), pltpu.VMEM((ch, 128), F32),
                       pltpu.VMEM_SHARED((npad, 128), F32),
                       pltpu.SemaphoreType.DMA, pltpu.SemaphoreType.DMA],
    )
    def k(h_hbm, r_hbm, c_hbm, z_hbm, out_hbm, riv, civ, vv0, vv1, acc, sg0, sg1):
        cid = lax.axis_index("c")
        sid = lax.axis_index("s")
        sems = (sg0, sg1)
        vvs = (vv0, vv1)
        wid = sid * 2 + cid
        pltpu.sync_copy(z_hbm.at[pl.ds(sid * rps, rps)], acc.at[pl.ds(sid * rps, rps)])
        pltpu.sync_copy(r_hbm.at[pl.ds(wid * per_w, per_w)], riv)
        pltpu.sync_copy(c_hbm.at[pl.ds(wid * nchunk, nchunk)], civ)
        plsc.subcore_barrier()

        def fire(j, b):
            ri = riv.at[pl.ds(j * ch, ch)]
            pltpu.async_copy(h_hbm.at[ri], vvs[b], sems[b])

        def drain_add(j, b):
            ri = riv.at[pl.ds(j * ch, ch)]
            pltpu.make_async_copy(h_hbm.at[ri], vvs[b], sems[b]).wait()
            pltpu.sync_copy(vvs[b], acc.at[civ.at[j]], add=True)

        fire(0, 0)

        def body(st, _):
            j0 = 2 * st
            fire(j0 + 1, 1)
            drain_add(j0, 0)

            @pl.when(j0 + 2 < nchunk)
            def _():
                fire(j0 + 2, 0)

            drain_add(j0 + 1, 1)
            return 0

        lax.fori_loop(0, nchunk // 2, body, 0)
        plsc.subcore_barrier()
        pltpu.sync_copy(acc.at[pl.ds(sid * rps, rps)],
                        out_hbm.at[pl.ds(cid * npad + sid * rps, rps)])

    return k(h, row, col2d, zeros)


def _coord_body(hr_ref, hc_ref, rel_ref,
                w1a_ref, w1b_ref, w1c_ref, b1_ref, w2_ref, b2_ref,
                w3_ref, b3_ref, ew1_ref, ewb1_ref, ew2r_ref, ewb2_ref,
                scale_ref, df_ref, db_ref):
    hr = hr_ref[...]
    hc = hc_ref[...]
    rel = rel_ref[...]
    w1a = w1a_ref[...]
    w1b = w1b_ref[...]
    w1c = w1c_ref[...]
    b1 = b1_ref[...]
    ha = _dot(hr, w1a)
    hb = _dot(hc, w1b)
    hab = _dot(hc, w1a)
    hbb = _dot(hr, w1b)
    rc = _dot(rel, w1c)
    t1 = _silu(ha + hb + rc + b1)
    u1 = _silu(hab + hbb - rc + b1)
    t2 = _silu(_dot(t1, w2_ref[...]) + b2_ref[...])
    u2 = _silu(_dot(u1, w2_ref[...]) + b2_ref[...])
    raw_f = jnp.tanh(_dot(t2, w3_ref[...]) + b3_ref[...])
    raw_b = jnp.tanh(_dot(u2, w3_ref[...]) + b3_ref[...])
    edge_len = jnp.sqrt(jnp.sum(rel * rel, axis=-1, keepdims=True))
    s = jnp.clip(scale_ref[0, 0], 0.0, 5.0)
    nf = jnp.maximum(jnp.sqrt(jnp.sum(raw_f * raw_f, axis=-1, keepdims=True)), 1e-8)
    nb_ = jnp.maximum(jnp.sqrt(jnp.sum(raw_b * raw_b, axis=-1, keepdims=True)), 1e-8)
    ew1 = ew1_ref[...]
    ewb1 = ewb1_ref[...]
    ew2r = ew2r_ref[...]
    ewb2 = ewb2_ref[0, 0]
    ew_f = _sigmoid(jnp.sum(_silu(_dot(rel, ew1) + ewb1) * ew2r, axis=-1, keepdims=True) + ewb2)
    ew_b = _sigmoid(jnp.sum(_silu(_dot(-rel, ew1) + ewb1) * ew2r, axis=-1, keepdims=True) + ewb2)
    common = 0.05 * s * edge_len
    df_ref[...] = raw_f / nf * (common * ew_f)
    db_ref[...] = raw_b / nb_ * (common * ew_b)


def _coord_pass(hr, hc, rel, wp, eb):
    epad = hr.shape[0]
    nb = epad // eb
    ebspec = lambda d: pl.BlockSpec((eb, d), lambda i: (i, 0))
    wspec = lambda a: pl.BlockSpec(a.shape, lambda i: (0,) * a.ndim)
    weights = (wp['cw1a'], wp['cw1b'], wp['cw1c'], wp['cb1'], wp['cw2'], wp['cb2'],
               wp['cw3'], wp['cb3'], wp['ew1'], wp['ewb1'], wp['ew2r'], wp['ewb2'],
               wp['scale'])
    return pl.pallas_call(
        _coord_body,
        grid=(nb,),
        in_specs=[ebspec(128), ebspec(128), ebspec(128)] +
                 [wspec(a) for a in weights],
        out_specs=(ebspec(128), ebspec(128)),
        out_shape=(jax.ShapeDtypeStruct((epad, 128), F32),
                   jax.ShapeDtypeStruct((epad, 128), F32)),
    )(hr, hc, rel, *weights)


def _msg_body(hr_ref, hc_ref, rel_ref,
              w1a_ref, w1b_ref, w1c_ref, b1_ref, g1_ref, be1_ref,
              w2_ref, b2_ref, g2_ref, be2_ref, m_ref):
    rel = rel_ref[...]
    m1 = _silu(_dot(hr_ref[...], w1a_ref[...]) + _dot(hc_ref[...], w1b_ref[...]) +
               _dot(rel, w1c_ref[...]) + b1_ref[...])
    m1 = _ln(m1, g1_ref[...], be1_ref[...])
    m2 = _silu(_dot(m1, w2_ref[...]) + b2_ref[...])
    m_ref[...] = _ln(m2, g2_ref[...], be2_ref[...])


def _msg_pass(hr, hc, rel2, wp, eb):
    epad = hr.shape[0]
    nb = epad // eb
    ebspec = lambda d: pl.BlockSpec((eb, d), lambda i: (i, 0))
    wspec = lambda a: pl.BlockSpec(a.shape, lambda i: (0,) * a.ndim)
    weights = (wp['mw1a'], wp['mw1b'], wp['mw1c'], wp['mb1'], wp['ln1g'], wp['ln1b'],
               wp['mw2'], wp['mb2'], wp['ln2g'], wp['ln2b'])
    return pl.pallas_call(
        _msg_body,
        grid=(nb,),
        in_specs=[ebspec(128), ebspec(128), ebspec(128)] +
                 [wspec(a) for a in weights],
        out_specs=ebspec(128),
        out_shape=jax.ShapeDtypeStruct((epad, 128), F32),
    )(hr, hc, rel2, *weights)


def _node_body(h_ref, agg_ref, w1a_ref, w1b_ref, b1_ref, lg_ref, lb_ref,
               w2_ref, b2_ref, g_ref, be_ref, out_ref):
    h = h_ref[...]
    npad = h_ref.shape[0]
    agg = agg_ref[pl.ds(0, npad), :] + agg_ref[pl.ds(npad, npad), :]
    nm = _silu(_dot(h, w1a_ref[...]) + _dot(agg, w1b_ref[...]) + b1_ref[...])
    nm = _ln(nm, lg_ref[...], lb_ref[...])
    nm = _dot(nm, w2_ref[...]) + b2_ref[...]
    out_ref[...] = _ln(h + nm, g_ref[...], be_ref[...])


def _node_pass(h, agg, wp):
    npad = h.shape[0]
    weights = (wp['nw1a'], wp['nw1b'], wp['nb1'], wp['nlng'], wp['nlnb'],
               wp['nw2'], wp['nb2'], wp['lng'], wp['lnb'])
    return pl.pallas_call(
        _node_body,
        out_shape=jax.ShapeDtypeStruct((npad, 128), F32),
    )(h, agg, *weights)


def _gc_body(agg_ref, h_ref, rw_ref, rb_ref, rootw_ref, out_ref):
    npad = h_ref.shape[0]
    agg = agg_ref[pl.ds(0, npad), :] + agg_ref[pl.ds(npad, npad), :]
    out_ref[...] = (_dot(agg, rw_ref[...]) + rb_ref[...] +
                    _dot(h_ref[...], rootw_ref[...]))


def _gc_pass(agg, h, rw, rb, rootw):
    npad = h.shape[0]
    return pl.pallas_call(
        _gc_body,
        out_shape=jax.ShapeDtypeStruct((npad, 128), F32),
    )(agg, h, rw, rb, rootw)


def _final_body(n_real, n_groups, h_ref, batch_ref,
                gw1_ref, gb1_ref, glg_ref, glb_ref, gw2_ref, gb2_ref,
                gw3r_ref, gb3_ref, ow1_ref, ob1_ref, olg_ref, olb_ref,
                ow2_ref, ob2_ref, out_ref):
    npad = h_ref.shape[0]
    h = h_ref[...]
    gate = _dot(h, gw1_ref[...]) + gb1_ref[...]
    gate = _gelu(_ln(gate, glg_ref[...], glb_ref[...]))
    gate = _gelu(_dot(gate, gw2_ref[...]) + gb2_ref[...])
    gate_s = jnp.sum(gate * gw3r_ref[...], axis=-1, keepdims=True) + gb3_ref[0, 0]
    valid = lax.broadcasted_iota(jnp.int32, (npad, 1), 0) < n_real
    gid = lax.broadcasted_iota(jnp.int32, (1, n_groups), 1)
    oh = jnp.logical_and(batch_ref[...] == gid, valid).astype(F32)
    gmax = jnp.max(jnp.where(oh > 0, gate_s, -1e30), axis=0, keepdims=True)
    gmax_g = jnp.sum(oh * gmax, axis=-1, keepdims=True)
    ex = jnp.where(valid, jnp.exp(gate_s - gmax_g), 0.0)
    den = jnp.sum(oh * ex, axis=0, keepdims=True)
    den_g = jnp.sum(oh * den, axis=-1, keepdims=True)
    attn = ex / jnp.maximum(den_g, 1e-16)
    pooled = lax.dot_general(oh, attn * h, (((0,), (0,)), ((), ())),
                             preferred_element_type=F32)
    o = _gelu(_ln(_dot(pooled, ow1_ref[...]) + ob1_ref[...], olg_ref[...], olb_ref[...]))
    out_ref[...] = _dot(o, ow2_ref[...]) + ob2_ref[...]


def _final_pass(h, batchp, wp, n_real, n_groups, out_dim):
    weights = (wp['gw1'], wp['gb1'], wp['glg'], wp['glb'], wp['gw2'], wp['gb2'],
               wp['gw3r'], wp['gb3'], wp['ow1'], wp['ob1'], wp['olg'], wp['olb'],
               wp['ow2'], wp['ob2'])
    return pl.pallas_call(
        functools.partial(_final_body, n_real, n_groups),
        out_shape=jax.ShapeDtypeStruct((n_groups, out_dim), F32),
    )(h, batchp, *weights)


def _prep_egcl(p):
    pad_rows = lambda w, r: jnp.concatenate(
        [w, jnp.zeros((r - w.shape[0], w.shape[1]), F32)], axis=0)
    pad_cols = lambda w, c: jnp.concatenate(
        [w, jnp.zeros((w.shape[0], c - w.shape[1]), F32)], axis=1)
    row = lambda v: v.reshape(1, -1)
    return {
        'cw1a': p['coord_w1'][:128], 'cw1b': p['coord_w1'][128:256],
        'cw1c': pad_rows(p['coord_w1'][256:], 128), 'cb1': row(p['coord_b1']),
        'cw2': p['coord_w2'], 'cb2': row(p['coord_b2']),
        'cw3': pad_cols(p['coord_w3'], 128), 'cb3': pad_cols(row(p['coord_b3']), 128),
        'ew1': pad_rows(p['ew_w1'], 128), 'ewb1': row(p['ew_b1']),
        'ew2r': p['ew_w2'].reshape(1, -1), 'ewb2': p['ew_b2'].reshape(1, 1),
        'scale': p['scale'].reshape(1, 1),
        'mw1a': p['edge_w1'][:128], 'mw1b': p['edge_w1'][128:256],
        'mw1c': pad_rows(p['edge_w1'][256:], 128), 'mb1': row(p['edge_b1']),
        'ln1g': row(p['edge_ln1_g']), 'ln1b': row(p['edge_ln1_b']),
        'mw2': p['edge_w2'], 'mb2': row(p['edge_b2']),
        'ln2g': row(p['edge_ln2_g']), 'ln2b': row(p['edge_ln2_b']),
        'nw1a': p['node_w1'][:128], 'nw1b': p['node_w1'][128:],
        'nb1': row(p['node_b1']), 'nlng': row(p['node_ln_g']),
        'nlnb': row(p['node_ln_b']), 'nw2': p['node_w2'], 'nb2': row(p['node_b2']),
        'lng': row(p['ln_g']), 'lnb': row(p['ln_b']),
    }


def _egcl_layer(h, posn, row, col, row2d, col2d, zeros, wp, eb, npad):
    hr, hc, rel = _sc_gather4(h, posn, row, col, zeros)
    df, db = _coord_pass(hr, hc, rel, wp, eb)
    pos2 = _sc_scatter_pos(posn, df, db, row, col)
    rel2 = _sc_gather_rel2(pos2, row, col, zeros)
    m = _msg_pass(hr, hc, rel2, wp, eb)
    agg = _sc_scatter_msg(m, row2d, zeros)
    h2 = _node_pass(h, agg, wp)
    return h2, pos2


def kernel(x, pos, edge_index, batch, params):
    n, in_dim = x.shape
    e = edge_index.shape[1]
    g = 16
    eb = 2048
    npad = _rup(n + 1, 128)
    epad = _rup(e, 4096)
    nb = epad // eb

    xp = jnp.zeros((npad, 16), F32).at[:n, :in_dim].set(x)
    posp = jnp.zeros((npad, 128), F32).at[:n, :3].set(pos)
    row = jnp.full((epad,), n, jnp.int32).at[:e].set(edge_index[0])
    col = jnp.full((epad,), n, jnp.int32).at[:e].set(edge_index[1])
    row2d = row.reshape(epad // 128, 128)
    col2d = col.reshape(epad // 128, 128)
    zeros = jnp.zeros((npad, 128), F32)
    batchp = jnp.full((npad, 1), g, jnp.int32).at[:n, 0].set(batch)

    p = params
    projw = jnp.concatenate([p['proj_w'], jnp.zeros((16 - in_dim, 128), F32)], axis=0)
    w0 = _prep_egcl(p['egcl0'])
    w2 = _prep_egcl(p['egcl2'])
    fin = {
        'gw1': p['gate_w1'], 'gb1': p['gate_b1'].reshape(1, -1),
        'glg': p['gate_ln_g'].reshape(1, -1), 'glb': p['gate_ln_b'].reshape(1, -1),
        'gw2': p['gate_w2'], 'gb2': p['gate_b2'].reshape(1, -1),
        'gw3r': p['gate_w3'].reshape(1, -1), 'gb3': p['gate_b3'].reshape(1, 1),
        'ow1': p['out_w1'], 'ob1': p['out_b1'].reshape(1, -1),
        'olg': p['out_ln_g'].reshape(1, -1), 'olb': p['out_ln_b'].reshape(1, -1),
        'ow2': p['out_w2'], 'ob2': p['out_b2'].reshape(1, -1),
    }

    h, posn = _prep(xp, posp, projw, p['proj_b'].reshape(1, -1),
                    p['proj_ln_g'].reshape(1, -1), p['proj_ln_b'].reshape(1, -1), n)

    h, pos2 = _egcl_layer(h, posn, row, col, row2d, col2d, zeros, w0, eb, npad)
    posn = _center(pos2, n)

    agg = _sc_gc(h, row, col2d, zeros)
    h = _gc_pass(agg, h, p['gc1']['rel_w'], p['gc1']['rel_b'].reshape(1, -1),
                 p['gc1']['root_w'])

    h, pos2 = _egcl_layer(h, posn, row, col, row2d, col2d, zeros, w2, eb, npad)

    agg = _sc_gc(h, row, col2d, zeros)
    h = _gc_pass(agg, h, p['gc3']['rel_w'], p['gc3']['rel_b'].reshape(1, -1),
                 p['gc3']['root_w'])

    out_dim = p['out_w2'].shape[1]
    return _final_pass(h, batchp, fin, n, g, out_dim)


# speedup vs baseline: 1.0405x; 1.0017x over previous
import functools

import jax
import jax.numpy as jnp
from jax import lax
from jax.experimental import pallas as pl
from jax.experimental.pallas import tpu as pltpu
from jax.experimental.pallas import tpu_sc as plsc

F32 = jnp.float32
_CH = 128


def _sc_mesh():
    return plsc.VectorSubcoreMesh(core_axis_name="c", subcore_axis_name="s")


def _rup(x, m):
    return (x + m - 1) // m * m


def _silu(x):
    return x * (1.0 / (1.0 + jnp.exp(-x)))


def _sigmoid(x):
    return 1.0 / (1.0 + jnp.exp(-x))


def _gelu(x):
    return 0.5 * x * (1.0 + lax.erf(x * 0.7071067811865476))


def _ln(x, g, b, eps=1e-5):
    m = jnp.mean(x, axis=-1, keepdims=True)
    v = jnp.mean((x - m) ** 2, axis=-1, keepdims=True)
    return (x - m) * lax.rsqrt(v + eps) * g + b


def _dot(a, b):
    return jnp.dot(a, b, preferred_element_type=F32)


def _prep_body(n_real, x_ref, p_ref, w_ref, b_ref, g_ref, beta_ref, h_ref, pn_ref):
    npad = p_ref.shape[0]
    valid = (lax.broadcasted_iota(jnp.int32, (npad, 1), 0) < n_real).astype(F32)
    pos = p_ref[...]
    mean = jnp.sum(pos, axis=0, keepdims=True) * (1.0 / n_real)
    cen = (pos - mean) * valid
    rms = jnp.sqrt(jnp.sum(cen * cen) * (1.0 / n_real))
    pn_ref[...] = cen * (1.0 / jnp.maximum(rms, 1e-6))
    y = _dot(x_ref[...], w_ref[...]) + b_ref[...]
    h_ref[...] = _gelu(_ln(y, g_ref[...], beta_ref[...]))


def _prep(xp, posp, w, b, g, beta, n_real):
    npad = xp.shape[0]
    return pl.pallas_call(
        functools.partial(_prep_body, n_real),
        out_shape=(jax.ShapeDtypeStruct((npad, 128), F32),
                   jax.ShapeDtypeStruct((npad, 128), F32)),
    )(xp, posp, w, b, g, beta)


def _center_body(n_real, p_ref, o_ref):
    npad = p_ref.shape[0]
    valid = (lax.broadcasted_iota(jnp.int32, (npad, 1), 0) < n_real).astype(F32)
    pos = p_ref[...]
    mean = jnp.sum(pos * valid, axis=0, keepdims=True) * (1.0 / n_real)
    cen = (pos - mean) * valid
    rms = jnp.sqrt(jnp.sum(cen * cen) * (1.0 / n_real))
    o_ref[...] = cen * (1.0 / jnp.maximum(rms, 1e-6))


def _center(pos, n_real):
    return pl.pallas_call(
        functools.partial(_center_body, n_real),
        out_shape=jax.ShapeDtypeStruct(pos.shape, F32),
    )(pos)


def _sc_gather4(h, pos, row, col, zeros):
    npad = h.shape[0]
    epad = row.shape[0]
    ch = 64
    per_w = epad // 32
    nchunk = per_w // ch
    assert nchunk % 3 == 2

    @functools.partial(
        pl.kernel,
        out_type=(jax.ShapeDtypeStruct((epad, 128), F32),
                  jax.ShapeDtypeStruct((epad, 128), F32),
                  jax.ShapeDtypeStruct((epad, 128), F32)),
        mesh=_sc_mesh(),
        scratch_types=[pltpu.VMEM((per_w,), jnp.int32), pltpu.VMEM((per_w,), jnp.int32),
                       pltpu.VMEM((3, ch, 128), F32), pltpu.VMEM((3, ch, 128), F32),
                       pltpu.VMEM((3, ch, 128), F32), pltpu.VMEM((3, ch, 128), F32),
                       pltpu.VMEM((ch, 128), F32),
                       pltpu.SemaphoreType.DMA, pltpu.SemaphoreType.DMA,
                       pltpu.SemaphoreType.DMA],
    )
    def k(h_hbm, p_hbm, r_hbm, c_hbm, z_hbm, hr_hbm, hc_hbm, rel_hbm,
          riv, civ, hrv, hcv, prv, pcv, relv, sg0, sg1, sg2):
        wid = lax.axis_index("s") * 2 + lax.axis_index("c")
        base = wid * per_w
        sems = (sg0, sg1, sg2)

        pltpu.sync_copy(r_hbm.at[pl.ds(base, per_w)], riv)
        pltpu.sync_copy(c_hbm.at[pl.ds(base, per_w)], civ)
        pltpu.sync_copy(z_hbm.at[pl.ds(0, ch)], relv)

        def fire(j, b):
            ri = riv.at[pl.ds(j * ch, ch)]
            ci = civ.at[pl.ds(j * ch, ch)]
            pltpu.async_copy(h_hbm.at[ri], hrv.at[b], sems[b])
            pltpu.async_copy(h_hbm.at[ci], hcv.at[b], sems[b])
            pltpu.async_copy(p_hbm.at[ri], prv.at[b], sems[b])
            pltpu.async_copy(p_hbm.at[ci], pcv.at[b], sems[b])

        def drain_write(j, b):
            ri = riv.at[pl.ds(j * ch, ch)]
            ci = civ.at[pl.ds(j * ch, ch)]
            pltpu.make_async_copy(h_hbm.at[ri], hrv.at[b], sems[b]).wait()
            pltpu.make_async_copy(h_hbm.at[ci], hcv.at[b], sems[b]).wait()
            pltpu.make_async_copy(p_hbm.at[ri], prv.at[b], sems[b]).wait()
            pltpu.make_async_copy(p_hbm.at[ci], pcv.at[b], sems[b]).wait()

            def rel_row(q, _):
                for u in range(4):
                    i = q * 4 + u
                    relv[i, pl.ds(0, 16)] = (prv[b, i, pl.ds(0, 16)] -
                                             pcv[b, i, pl.ds(0, 16)])
                return 0

            lax.fori_loop(0, ch // 4, rel_row, 0)
            off = base + j * ch
            pltpu.sync_copy(hrv.at[b], hr_hbm.at[pl.ds(off, ch)])
            pltpu.sync_copy(hcv.at[b], hc_hbm.at[pl.ds(off, ch)])
            pltpu.sync_copy(relv, rel_hbm.at[pl.ds(off, ch)])

        fire(0, 0)
        fire(1, 1)

        def body(st, _):
            j0 = 3 * st
            fire(j0 + 2, 2)
            drain_write(j0, 0)
            fire(j0 + 3, 0)
            drain_write(j0 + 1, 1)
            fire(j0 + 4, 1)
            drain_write(j0 + 2, 2)
            return 0

        lax.fori_loop(0, (nchunk - 2) // 3, body, 0)
        drain_write(nchunk - 2, 0)
        drain_write(nchunk - 1, 1)

    return k(h, pos, row, col, zeros)


def _sc_gather_rel2(pos, row, col, zeros):
    epad = row.shape[0]
    ch = 64
    per_w = epad // 32
    nchunk = per_w // ch
    assert nchunk % 3 == 2

    @functools.partial(
        pl.kernel,
        out_type=jax.ShapeDtypeStruct((epad, 128), F32),
        mesh=_sc_mesh(),
        scratch_types=[pltpu.VMEM((per_w,), jnp.int32), pltpu.VMEM((per_w,), jnp.int32),
                       pltpu.VMEM((3, ch, 128), F32), pltpu.VMEM((3, ch, 128), F32),
                       pltpu.VMEM((ch, 128), F32),
                       pltpu.SemaphoreType.DMA, pltpu.SemaphoreType.DMA,
                       pltpu.SemaphoreType.DMA],
    )
    def k(p_hbm, r_hbm, c_hbm, z_hbm, rel_hbm, riv, civ, prv, pcv, relv,
          sg0, sg1, sg2):
        wid = lax.axis_index("s") * 2 + lax.axis_index("c")
        base = wid * per_w
        sems = (sg0, sg1, sg2)

        pltpu.sync_copy(r_hbm.at[pl.ds(base, per_w)], riv)
        pltpu.sync_copy(c_hbm.at[pl.ds(base, per_w)], civ)
        pltpu.sync_copy(z_hbm.at[pl.ds(0, ch)], relv)

        def fire(j, b):
            ri = riv.at[pl.ds(j * ch, ch)]
            ci = civ.at[pl.ds(j * ch, ch)]
            pltpu.async_copy(p_hbm.at[ri], prv.at[b], sems[b])
            pltpu.async_copy(p_hbm.at[ci], pcv.at[b], sems[b])

        def drain_write(j, b):
            ri = riv.at[pl.ds(j * ch, ch)]
            ci = civ.at[pl.ds(j * ch, ch)]
            pltpu.make_async_copy(p_hbm.at[ri], prv.at[b], sems[b]).wait()
            pltpu.make_async_copy(p_hbm.at[ci], pcv.at[b], sems[b]).wait()

            def rel_row(q, _):
                for u in range(4):
                    i = q * 4 + u
                    relv[i, pl.ds(0, 16)] = (prv[b, i, pl.ds(0, 16)] -
                                             pcv[b, i, pl.ds(0, 16)])
                return 0

            lax.fori_loop(0, ch // 4, rel_row, 0)
            off = base + j * ch
            pltpu.sync_copy(relv, rel_hbm.at[pl.ds(off, ch)])

        fire(0, 0)
        fire(1, 1)

        def body(st, _):
            j0 = 3 * st
            fire(j0 + 2, 2)
            drain_write(j0, 0)
            fire(j0 + 3, 0)
            drain_write(j0 + 1, 1)
            fire(j0 + 4, 1)
            drain_write(j0 + 2, 2)
            return 0

        lax.fori_loop(0, (nchunk - 2) // 3, body, 0)
        drain_write(nchunk - 2, 0)
        drain_write(nchunk - 1, 1)

    return k(pos, row, col, zeros)


def _sc_scatter_pos(init, df, db, row, col):
    npad = init.shape[0]
    epad = row.shape[0]
    ch = _CH
    per_w = epad // 16
    nchunk = per_w // ch
    rps = npad // 16

    @functools.partial(
        pl.kernel,
        out_type=jax.ShapeDtypeStruct((npad, 128), F32),
        mesh=_sc_mesh(),
        scratch_types=[pltpu.VMEM((ch,), jnp.int32),
                       pltpu.VMEM((ch, 128), F32), pltpu.VMEM((ch, 128), F32),
                       pltpu.VMEM_SHARED((npad, 128), F32),
                       pltpu.SemaphoreType.DMA, pltpu.SemaphoreType.DMA],
    )
    def k(init_hbm, df_hbm, db_hbm, r_hbm, c_hbm, out_hbm,
          iv, vv0, vv1, acc, sg0, sg1):
        cid = lax.axis_index("c")
        sid = lax.axis_index("s")
        sems = (sg0, sg1)
        vvs = (vv0, vv1)

        @pl.when(cid == 0)
        def _():
            pltpu.sync_copy(init_hbm.at[pl.ds(sid * rps, rps)],
                            acc.at[pl.ds(sid * rps, rps)])
            plsc.subcore_barrier()
            base = sid * per_w

            def one_pass(v_hbm, i_hbm):
                def fire(j, b):
                    pltpu.async_copy(v_hbm.at[pl.ds(base + j * ch, ch)], vvs[b],
                                     sems[b])

                def drain_add(j, b):
                    pltpu.sync_copy(i_hbm.at[pl.ds(base + j * ch, ch)], iv)
                    pltpu.make_async_copy(v_hbm.at[pl.ds(base + j * ch, ch)],
                                          vvs[b], sems[b]).wait()
                    pltpu.sync_copy(vvs[b], acc.at[iv], add=True)

                fire(0, 0)

                def body(st, _):
                    j0 = 2 * st
                    fire(j0 + 1, 1)
                    drain_add(j0, 0)

                    @pl.when(j0 + 2 < nchunk)
                    def _():
                        fire(j0 + 2, 0)

                    drain_add(j0 + 1, 1)
                    return 0

                lax.fori_loop(0, nchunk // 2, body, 0)

            one_pass(df_hbm, r_hbm)
            one_pass(db_hbm, c_hbm)
            plsc.subcore_barrier()
            pltpu.sync_copy(acc.at[pl.ds(sid * rps, rps)],
                            out_hbm.at[pl.ds(sid * rps, rps)])

    return k(init, df, db, row, col)


def _sc_scatter_msg(m, row2d, zeros):
    npad = zeros.shape[0]
    ch = _CH
    nrow = row2d.shape[0]
    nchunk = nrow // 32
    per_w = nchunk * ch
    rps = npad // 16

    @functools.partial(
        pl.kernel,
        out_type=jax.ShapeDtypeStruct((2 * npad, 128), F32),
        mesh=_sc_mesh(),
        scratch_types=[pltpu.VMEM((nchunk, ch), jnp.int32),
                       pltpu.VMEM((ch, 128), F32), pltpu.VMEM((ch, 128), F32),
                       pltpu.VMEM_SHARED((npad, 128), F32),
                       pltpu.SemaphoreType.DMA, pltpu.SemaphoreType.DMA],
    )
    def k(m_hbm, r_hbm, z_hbm, out_hbm, riv, vv0, vv1, acc, sg0, sg1):
        cid = lax.axis_index("c")
        sid = lax.axis_index("s")
        sems = (sg0, sg1)
        vvs = (vv0, vv1)
        wid = sid * 2 + cid
        pltpu.sync_copy(z_hbm.at[pl.ds(sid * rps, rps)], acc.at[pl.ds(sid * rps, rps)])
        pltpu.sync_copy(r_hbm.at[pl.ds(wid * nchunk, nchunk)], riv)
        plsc.subcore_barrier()
        base = wid * per_w

        def fire(j, b):
            pltpu.async_copy(m_hbm.at[pl.ds(base + j * ch, ch)], vvs[b], sems[b])

        def drain_add(j, b):
            pltpu.make_async_copy(m_hbm.at[pl.ds(base + j * ch, ch)], vvs[b],
                                  sems[b]).wait()
            pltpu.sync_copy(vvs[b], acc.at[riv.at[j]], add=True)

        fire(0, 0)

        def body(st, _):
            j0 = 2 * st
            fire(j0 + 1, 1)
            drain_add(j0, 0)

            @pl.when(j0 + 2 < nchunk)
            def _():
                fire(j0 + 2, 0)

            drain_add(j0 + 1, 1)
            return 0

        lax.fori_loop(0, nchunk // 2, body, 0)
        plsc.subcore_barrier()
        pltpu.sync_copy(acc.at[pl.ds(sid * rps, rps)],
                        out_hbm.at[pl.ds(cid * npad + sid * rps, rps)])

    return k(m, row2d, zeros)


def _sc_gc(h, row, col2d, zeros):
    npad = h.shape[0]
    epad = row.shape[0]
    ch = _CH
    per_w = epad // 32
    nchunk = per_w // ch
    rps = npad // 16

    @functools.partial(
        pl.kernel,
        out_type=jax.ShapeDtypeStruct((2 * npad, 128), F32),
        mesh=_sc_mesh(),
        scratch_types=[pltpu.VMEM((per_w,), jnp.int32),
                       pltpu.VMEM((nchunk, ch), jnp.int32),
                       pltpu.VMEM((ch, 128), F32), pltpu.VMEM((ch, 128), F32),
                       pltpu.VMEM_SHARED((npad, 128), F32),
                       pltpu.SemaphoreType.DMA, pltpu.SemaphoreType.DMA],
    )
    def k(h_hbm, r_hbm, c_hbm, z_hbm, out_hbm, riv, civ, vv0, vv1, acc, sg0, sg1):
        cid = lax.axis_index("c")
        sid = lax.axis_index("s")
        sems = (sg0, sg1)
        vvs = (vv0, vv1)
        wid = sid * 2 + cid
        pltpu.sync_copy(z_hbm.at[pl.ds(sid * rps, rps)], acc.at[pl.ds(sid * rps, rps)])
        pltpu.sync_copy(r_hbm.at[pl.ds(wid * per_w, per_w)], riv)
        pltpu.sync_copy(c_hbm.at[pl.ds(wid * nchunk, nchunk)], civ)
        plsc.subcore_barrier()

        def fire(j, b):
            ri = riv.at[pl.ds(j * ch, ch)]
            pltpu.async_copy(h_hbm.at[ri], vvs[b], sems[b])

        def drain_add(j, b):
            ri = riv.at[pl.ds(j * ch, ch)]
            pltpu.make_async_copy(h_hbm.at[ri], vvs[b], sems[b]).wait()
            pltpu.sync_copy(vvs[b], acc.at[civ.at[j]], add=True)

        fire(0, 0)

        def body(st, _):
            j0 = 2 * st
            fire(j0 + 1, 1)
            drain_add(j0, 0)

            @pl.when(j0 + 2 < nchunk)
            def _():
                fire(j0 + 2, 0)

            drain_add(j0 + 1, 1)
            return 0

        lax.fori_loop(0, nchunk // 2, body, 0)
        plsc.subcore_barrier()
        pltpu.sync_copy(acc.at[pl.ds(sid * rps, rps)],
                        out_hbm.at[pl.ds(cid * npad + sid * rps, rps)])

    return k(h, row, col2d, zeros)


def _coord_body(hr_ref, hc_ref, rel_ref,
                w1a_ref, w1b_ref, w1c_ref, b1_ref, w2_ref, b2_ref,
                w3_ref, b3_ref, ew1_ref, ewb1_ref, ew2r_ref, ewb2_ref,
                scale_ref, df_ref, db_ref):
    hr = hr_ref[...]
    hc = hc_ref[...]
    rel = rel_ref[...]
    w1a = w1a_ref[...]
    w1b = w1b_ref[...]
    w1c = w1c_ref[...]
    b1 = b1_ref[...]
    ha = _dot(hr, w1a)
    hb = _dot(hc, w1b)
    hab = _dot(hc, w1a)
    hbb = _dot(hr, w1b)
    rc = _dot(rel, w1c)
    t1 = _silu(ha + hb + rc + b1)
    u1 = _silu(hab + hbb - rc + b1)
    t2 = _silu(_dot(t1, w2_ref[...]) + b2_ref[...])
    u2 = _silu(_dot(u1, w2_ref[...]) + b2_ref[...])
    raw_f = jnp.tanh(_dot(t2, w3_ref[...]) + b3_ref[...])
    raw_b = jnp.tanh(_dot(u2, w3_ref[...]) + b3_ref[...])
    edge_len = jnp.sqrt(jnp.sum(rel * rel, axis=-1, keepdims=True))
    s = jnp.clip(scale_ref[0, 0], 0.0, 5.0)
    nf = jnp.maximum(jnp.sqrt(jnp.sum(raw_f * raw_f, axis=-1, keepdims=True)), 1e-8)
    nb_ = jnp.maximum(jnp.sqrt(jnp.sum(raw_b * raw_b, axis=-1, keepdims=True)), 1e-8)
    ew1 = ew1_ref[...]
    ewb1 = ewb1_ref[...]
    ew2r = ew2r_ref[...]
    ewb2 = ewb2_ref[0, 0]
    ew_f = _sigmoid(jnp.sum(_silu(_dot(rel, ew1) + ewb1) * ew2r, axis=-1, keepdims=True) + ewb2)
    ew_b = _sigmoid(jnp.sum(_silu(_dot(-rel, ew1) + ewb1) * ew2r, axis=-1, keepdims=True) + ewb2)
    common = 0.05 * s * edge_len
    df_ref[...] = raw_f / nf * (common * ew_f)
    db_ref[...] = raw_b / nb_ * (common * ew_b)


def _coord_pass(hr, hc, rel, wp, eb):
    epad = hr.shape[0]
    nb = epad // eb
    ebspec = lambda d: pl.BlockSpec((eb, d), lambda i: (i, 0))
    wspec = lambda a: pl.BlockSpec(a.shape, lambda i: (0,) * a.ndim)
    weights = (wp['cw1a'], wp['cw1b'], wp['cw1c'], wp['cb1'], wp['cw2'], wp['cb2'],
               wp['cw3'], wp['cb3'], wp['ew1'], wp['ewb1'], wp['ew2r'], wp['ewb2'],
               wp['scale'])
    return pl.pallas_call(
        _coord_body,
        grid=(nb,),
        in_specs=[ebspec(128), ebspec(128), ebspec(128)] +
                 [wspec(a) for a in weights],
        out_specs=(ebspec(128), ebspec(128)),
        out_shape=(jax.ShapeDtypeStruct((epad, 128), F32),
                   jax.ShapeDtypeStruct((epad, 128), F32)),
    )(hr, hc, rel, *weights)


def _msg_body(hr_ref, hc_ref, rel_ref,
              w1a_ref, w1b_ref, w1c_ref, b1_ref, g1_ref, be1_ref,
              w2_ref, b2_ref, g2_ref, be2_ref, m_ref):
    rel = rel_ref[...]
    m1 = _silu(_dot(hr_ref[...], w1a_ref[...]) + _dot(hc_ref[...], w1b_ref[...]) +
               _dot(rel, w1c_ref[...]) + b1_ref[...])
    m1 = _ln(m1, g1_ref[...], be1_ref[...])
    m2 = _silu(_dot(m1, w2_ref[...]) + b2_ref[...])
    m_ref[...] = _ln(m2, g2_ref[...], be2_ref[...])


def _msg_pass(hr, hc, rel2, wp, eb):
    epad = hr.shape[0]
    nb = epad // eb
    ebspec = lambda d: pl.BlockSpec((eb, d), lambda i: (i, 0))
    wspec = lambda a: pl.BlockSpec(a.shape, lambda i: (0,) * a.ndim)
    weights = (wp['mw1a'], wp['mw1b'], wp['mw1c'], wp['mb1'], wp['ln1g'], wp['ln1b'],
               wp['mw2'], wp['mb2'], wp['ln2g'], wp['ln2b'])
    return pl.pallas_call(
        _msg_body,
        grid=(nb,),
        in_specs=[ebspec(128), ebspec(128), ebspec(128)] +
                 [wspec(a) for a in weights],
        out_specs=ebspec(128),
        out_shape=jax.ShapeDtypeStruct((epad, 128), F32),
    )(hr, hc, rel2, *weights)


def _node_body(h_ref, agg_ref, w1a_ref, w1b_ref, b1_ref, lg_ref, lb_ref,
               w2_ref, b2_ref, g_ref, be_ref, out_ref):
    h = h_ref[...]
    npad = h_ref.shape[0]
    agg = agg_ref[pl.ds(0, npad), :] + agg_ref[pl.ds(npad, npad), :]
    nm = _silu(_dot(h, w1a_ref[...]) + _dot(agg, w1b_ref[...]) + b1_ref[...])
    nm = _ln(nm, lg_ref[...], lb_ref[...])
    nm = _dot(nm, w2_ref[...]) + b2_ref[...]
    out_ref[...] = _ln(h + nm, g_ref[...], be_ref[...])


def _node_pass(h, agg, wp):
    npad = h.shape[0]
    weights = (wp['nw1a'], wp['nw1b'], wp['nb1'], wp['nlng'], wp['nlnb'],
               wp['nw2'], wp['nb2'], wp['lng'], wp['lnb'])
    return pl.pallas_call(
        _node_body,
        out_shape=jax.ShapeDtypeStruct((npad, 128), F32),
    )(h, agg, *weights)


def _gc_body(agg_ref, h_ref, rw_ref, rb_ref, rootw_ref, out_ref):
    npad = h_ref.shape[0]
    agg = agg_ref[pl.ds(0, npad), :] + agg_ref[pl.ds(npad, npad), :]
    out_ref[...] = (_dot(agg, rw_ref[...]) + rb_ref[...] +
                    _dot(h_ref[...], rootw_ref[...]))


def _gc_pass(agg, h, rw, rb, rootw):
    npad = h.shape[0]
    return pl.pallas_call(
        _gc_body,
        out_shape=jax.ShapeDtypeStruct((npad, 128), F32),
    )(agg, h, rw, rb, rootw)


def _final_body(n_real, n_groups, h_ref, batch_ref,
                gw1_ref, gb1_ref, glg_ref, glb_ref, gw2_ref, gb2_ref,
                gw3r_ref, gb3_ref, ow1_ref, ob1_ref, olg_ref, olb_ref,
                ow2_ref, ob2_ref, out_ref):
    npad = h_ref.shape[0]
    h = h_ref[...]
    gate = _dot(h, gw1_ref[...]) + gb1_ref[...]
    gate = _gelu(_ln(gate, glg_ref[...], glb_ref[...]))
    gate = _gelu(_dot(gate, gw2_ref[...]) + gb2_ref[...])
    gate_s = jnp.sum(gate * gw3r_ref[...], axis=-1, keepdims=True) + gb3_ref[0, 0]
    valid = lax.broadcasted_iota(jnp.int32, (npad, 1), 0) < n_real
    gid = lax.broadcasted_iota(jnp.int32, (1, n_groups), 1)
    oh = jnp.logical_and(batch_ref[...] == gid, valid).astype(F32)
    gmax = jnp.max(jnp.where(oh > 0, gate_s, -1e30), axis=0, keepdims=True)
    gmax_g = jnp.sum(oh * gmax, axis=-1, keepdims=True)
    ex = jnp.where(valid, jnp.exp(gate_s - gmax_g), 0.0)
    den = jnp.sum(oh * ex, axis=0, keepdims=True)
    den_g = jnp.sum(oh * den, axis=-1, keepdims=True)
    attn = ex / jnp.maximum(den_g, 1e-16)
    pooled = lax.dot_general(oh, attn * h, (((0,), (0,)), ((), ())),
                             preferred_element_type=F32)
    o = _gelu(_ln(_dot(pooled, ow1_ref[...]) + ob1_ref[...], olg_ref[...], olb_ref[...]))
    out_ref[...] = _dot(o, ow2_ref[...]) + ob2_ref[...]


def _final_pass(h, batchp, wp, n_real, n_groups, out_dim):
    weights = (wp['gw1'], wp['gb1'], wp['glg'], wp['glb'], wp['gw2'], wp['gb2'],
               wp['gw3r'], wp['gb3'], wp['ow1'], wp['ob1'], wp['olg'], wp['olb'],
               wp['ow2'], wp['ob2'])
    return pl.pallas_call(
        functools.partial(_final_body, n_real, n_groups),
        out_shape=jax.ShapeDtypeStruct((n_groups, out_dim), F32),
    )(h, batchp, *weights)


def _prep_egcl(p):
    pad_rows = lambda w, r: jnp.concatenate(
        [w, jnp.zeros((r - w.shape[0], w.shape[1]), F32)], axis=0)
    pad_cols = lambda w, c: jnp.concatenate(
        [w, jnp.zeros((w.shape[0], c - w.shape[1]), F32)], axis=1)
    row = lambda v: v.reshape(1, -1)
    return {
        'cw1a': p['coord_w1'][:128], 'cw1b': p['coord_w1'][128:256],
        'cw1c': pad_rows(p['coord_w1'][256:], 128), 'cb1': row(p['coord_b1']),
        'cw2': p['coord_w2'], 'cb2': row(p['coord_b2']),
        'cw3': pad_cols(p['coord_w3'], 128), 'cb3': pad_cols(row(p['coord_b3']), 128),
        'ew1': pad_rows(p['ew_w1'], 128), 'ewb1': row(p['ew_b1']),
        'ew2r': p['ew_w2'].reshape(1, -1), 'ewb2': p['ew_b2'].reshape(1, 1),
        'scale': p['scale'].reshape(1, 1),
        'mw1a': p['edge_w1'][:128], 'mw1b': p['edge_w1'][128:256],
        'mw1c': pad_rows(p['edge_w1'][256:], 128), 'mb1': row(p['edge_b1']),
        'ln1g': row(p['edge_ln1_g']), 'ln1b': row(p['edge_ln1_b']),
        'mw2': p['edge_w2'], 'mb2': row(p['edge_b2']),
        'ln2g': row(p['edge_ln2_g']), 'ln2b': row(p['edge_ln2_b']),
        'nw1a': p['node_w1'][:128], 'nw1b': p['node_w1'][128:],
        'nb1': row(p['node_b1']), 'nlng': row(p['node_ln_g']),
        'nlnb': row(p['node_ln_b']), 'nw2': p['node_w2'], 'nb2': row(p['node_b2']),
        'lng': row(p['ln_g']), 'lnb': row(p['ln_b']),
    }


def _egcl_layer(h, posn, row, col, row2d, col2d, zeros, wp, eb, npad):
    hr, hc, rel = _sc_gather4(h, posn, row, col, zeros)
    df, db = _coord_pass(hr, hc, rel, wp, eb)
    pos2 = _sc_scatter_pos(posn, df, db, row, col)
    rel2 = _sc_gather_rel2(pos2, row, col, zeros)
    m = _msg_pass(hr, hc, rel2, wp, eb)
    agg = _sc_scatter_msg(m, row2d, zeros)
    h2 = _node_pass(h, agg, wp)
    return h2, pos2


def kernel(x, pos, edge_index, batch, params):
    n, in_dim = x.shape
    e = edge_index.shape[1]
    g = 16
    eb = 2048
    npad = _rup(n + 1, 128)
    epad = _rup(e, 4096)
    nb = epad // eb

    xp = jnp.zeros((npad, 16), F32).at[:n, :in_dim].set(x)
    posp = jnp.zeros((npad, 128), F32).at[:n, :3].set(pos)
    row = jnp.full((epad,), n, jnp.int32).at[:e].set(edge_index[0])
    col = jnp.full((epad,), n, jnp.int32).at[:e].set(edge_index[1])
    row2d = row.reshape(epad // 128, 128)
    col2d = col.reshape(epad // 128, 128)
    zeros = jnp.zeros((npad, 128), F32)
    batchp = jnp.full((npad, 1), g, jnp.int32).at[:n, 0].set(batch)

    p = params
    projw = jnp.concatenate([p['proj_w'], jnp.zeros((16 - in_dim, 128), F32)], axis=0)
    w0 = _prep_egcl(p['egcl0'])
    w2 = _prep_egcl(p['egcl2'])
    fin = {
        'gw1': p['gate_w1'], 'gb1': p['gate_b1'].reshape(1, -1),
        'glg': p['gate_ln_g'].reshape(1, -1), 'glb': p['gate_ln_b'].reshape(1, -1),
        'gw2': p['gate_w2'], 'gb2': p['gate_b2'].reshape(1, -1),
        'gw3r': p['gate_w3'].reshape(1, -1), 'gb3': p['gate_b3'].reshape(1, 1),
        'ow1': p['out_w1'], 'ob1': p['out_b1'].reshape(1, -1),
        'olg': p['out_ln_g'].reshape(1, -1), 'olb': p['out_ln_b'].reshape(1, -1),
        'ow2': p['out_w2'], 'ob2': p['out_b2'].reshape(1, -1),
    }

    h, posn = _prep(xp, posp, projw, p['proj_b'].reshape(1, -1),
                    p['proj_ln_g'].reshape(1, -1), p['proj_ln_b'].reshape(1, -1), n)

    h, pos2 = _egcl_layer(h, posn, row, col, row2d, col2d, zeros, w0, eb, npad)
    posn = _center(pos2, n)

    agg = _sc_gc(h, row, col2d, zeros)
    h = _gc_pass(agg, h, p['gc1']['rel_w'], p['gc1']['rel_b'].reshape(1, -1),
                 p['gc1']['root_w'])

    h, pos2 = _egcl_layer(h, posn, row, col, row2d, col2d, zeros, w2, eb, npad)

    agg = _sc_gc(h, row, col2d, zeros)
    h = _gc_pass(agg, h, p['gc3']['rel_w'], p['gc3']['rel_b'].reshape(1, -1),
                 p['gc3']['root_w'])

    out_dim = p['out_w2'].shape[1]
    return _final_pass(h, batchp, fin, n, g, out_dim)
